# baseline (speedup 1.0000x reference)
import functools

import jax
import jax.numpy as jnp
from jax import lax
from jax.experimental import pallas as pl
from jax.experimental.pallas import tpu as pltpu
from jax.experimental.pallas import tpu_sc as plsc

N = 2048
D = 768
H = 12
DH = D // H
FF = 3072
E = 8
K = 2
EPS = 1e-05

T = 512
S = N * K + E * T
NT = S // T
FB = 1024
NJ = FF // FB

NB = 256
QB = 512


def _qkv_body(x_ref, w_ref, b_ref, o_ref, ws_ref):
    @pl.when(pl.program_id(0) == 0)
    def _():
        ws_ref[...] = w_ref[...].astype(jnp.bfloat16)

    x = x_ref[...].astype(jnp.bfloat16)
    acc = lax.dot_general(x, ws_ref[...], (((1,), (1,)), ((), ())),
                          preferred_element_type=jnp.float32)
    acc = acc + b_ref[...]
    cio = lax.broadcasted_iota(jnp.int32, (1, 3 * D), 1)
    acc = acc * jnp.where(cio < D, 1.0 / (DH ** 0.5), 1.0)
    o_ref[...] = acc.astype(jnp.bfloat16)


def _qkv(src, in_proj_w, in_proj_b):
    return pl.pallas_call(
        _qkv_body,
        grid=(N // NB,),
        in_specs=[
            pl.BlockSpec((NB, D), lambda i: (i, 0)),
            pl.BlockSpec((3 * D, D), lambda i: (0, 0)),
            pl.BlockSpec((1, 3 * D), lambda i: (0, 0)),
        ],
        out_specs=pl.BlockSpec((NB, 3 * D), lambda i: (i, 0)),
        out_shape=jax.ShapeDtypeStruct((N, 3 * D), jnp.bfloat16),
        scratch_shapes=[pltpu.VMEM((3 * D, D), jnp.bfloat16)],
    )(src, in_proj_w, in_proj_b.reshape(1, 3 * D))


def _attn_body(q_ref, k_ref, v_ref, o_ref):
    q = q_ref[0]
    k = k_ref[0]
    s = lax.dot_general(q, k, (((1,), (1,)), ((), ())),
                        preferred_element_type=jnp.float32)
    m = jnp.max(s, axis=1, keepdims=True)
    p = jnp.exp(s - m)
    l = jnp.sum(p, axis=1, keepdims=True)
    o = lax.dot_general(p.astype(jnp.bfloat16), v_ref[0],
                        (((1,), (0,)), ((), ())),
                        preferred_element_type=jnp.float32)
    o_ref[0] = (o * (1.0 / l)).astype(jnp.bfloat16)


def _attention(qkvh):
    return pl.pallas_call(
        _attn_body,
        grid=(H, N // QB),
        in_specs=[
            pl.BlockSpec((1, QB, DH), lambda h, i: (h, i, 0)),
            pl.BlockSpec((1, N, DH), lambda h, i: (H + h, 0, 0)),
            pl.BlockSpec((1, N, DH), lambda h, i: (2 * H + h, 0, 0)),
        ],
        out_specs=pl.BlockSpec((1, QB, DH), lambda h, i: (h, i, 0)),
        out_shape=jax.ShapeDtypeStruct((H, N, DH), jnp.bfloat16),
    )(qkvh, qkvh, qkvh)


def _post_attn_body(o_ref, src_ref, wo_ref, bo_ref, n1w_ref, n1b_ref,
                    gw_ref, gb_ref, x1_ref, topi_ref, topw_ref, ws_ref):
    @pl.when(pl.program_id(0) == 0)
    def _():
        ws_ref[...] = wo_ref[...].astype(jnp.bfloat16)

    o = o_ref[...]
    sa = lax.dot_general(o, ws_ref[...], (((1,), (1,)), ((), ())),
                         preferred_element_type=jnp.float32)
    y = sa + bo_ref[...] + src_ref[...]
    mu = jnp.mean(y, axis=1, keepdims=True)
    yc = y - mu
    var = jnp.mean(yc * yc, axis=1, keepdims=True)
    x1 = yc * lax.rsqrt(var + EPS) * n1w_ref[...] + n1b_ref[...]
    x1_ref[...] = x1

    logits = lax.dot_general(x1, gw_ref[...], (((1,), (1,)), ((), ())),
                             preferred_element_type=jnp.float32) + gb_ref[...]
    lm = jnp.max(logits, axis=1, keepdims=True)
    eg = jnp.exp(logits - lm)
    g = eg / jnp.sum(eg, axis=1, keepdims=True)
    eio = lax.broadcasted_iota(jnp.int32, (NB, E), 1)
    m1 = jnp.max(g, axis=1, keepdims=True)
    i1 = jnp.min(jnp.where(g == m1, eio, E), axis=1, keepdims=True)
    g2 = jnp.where(eio == i1, -1.0, g)
    m2 = jnp.max(g2, axis=1, keepdims=True)
    i2 = jnp.min(jnp.where(g2 == m2, eio, E), axis=1, keepdims=True)
    topi_ref[...] = jnp.where(eio == 0, i1, jnp.where(eio == 1, i2, 0))
    topw_ref[...] = jnp.where(eio == 0, m1, jnp.where(eio == 1, m2, 0.0))


def _post_attn(o, src, out_proj_w, out_proj_b, norm1_w, norm1_b, gate_w, gate_b):
    return pl.pallas_call(
        _post_attn_body,
        grid=(N // NB,),
        in_specs=[
            pl.BlockSpec((NB, D), lambda i: (i, 0)),
            pl.BlockSpec((NB, D), lambda i: (i, 0)),
            pl.BlockSpec((D, D), lambda i: (0, 0)),
            pl.BlockSpec((1, D), lambda i: (0, 0)),
            pl.BlockSpec((1, D), lambda i: (0, 0)),
            pl.BlockSpec((1, D), lambda i: (0, 0)),
            pl.BlockSpec((E, D), lambda i: (0, 0)),
            pl.BlockSpec((1, E), lambda i: (0, 0)),
        ],
        out_specs=[
            pl.BlockSpec((NB, D), lambda i: (i, 0)),
            pl.BlockSpec((NB, E), lambda i: (i, 0)),
            pl.BlockSpec((NB, E), lambda i: (i, 0)),
        ],
        out_shape=[
            jax.ShapeDtypeStruct((N, D), jnp.float32),
            jax.ShapeDtypeStruct((N, E), jnp.int32),
            jax.ShapeDtypeStruct((N, E), jnp.float32),
        ],
        scratch_shapes=[pltpu.VMEM((D, D), jnp.bfloat16)],
    )(o, src, out_proj_w, out_proj_b.reshape(1, D), norm1_w.reshape(1, D),
      norm1_b.reshape(1, D), gate_w, gate_b.reshape(1, E))


def _rank_body(topi_ref, rank_ref, cnt_ref, tot_ref):
    i = pl.program_id(0)

    @pl.when(i == 0)
    def _():
        tot_ref[...] = jnp.zeros_like(tot_ref)

    eio = lax.broadcasted_iota(jnp.int32, (NB, E), 1)
    oh0 = (topi_ref[:, 0:1] == eio).astype(jnp.float32)
    oh1 = (topi_ref[:, 1:2] == eio).astype(jnp.float32)
    c = oh0 + oh1
    r = lax.broadcasted_iota(jnp.int32, (NB, NB), 0)
    cc = lax.broadcasted_iota(jnp.int32, (NB, NB), 1)
    strict_l = (r > cc).astype(jnp.float32)
    pre = lax.dot_general(strict_l, c, (((1,), (0,)), ((), ())),
                          preferred_element_type=jnp.float32)
    base = tot_ref[...] + pre
    r0 = jnp.sum(oh0 * base, axis=1, keepdims=True)
    r1 = jnp.sum(oh1 * base, axis=1, keepdims=True)
    eiof = lax.broadcasted_iota(jnp.int32, (NB, E), 1)
    rank_ref[...] = jnp.where(eiof == 0, r0, jnp.where(eiof == 1, r1, 0.0))
    tot_ref[...] = tot_ref[0:1, :] + jnp.sum(c, axis=0, keepdims=True)
    cnt_ref[...] = tot_ref[0:1, :]


def _ranks(topi):
    return pl.pallas_call(
        _rank_body,
        grid=(N // NB,),
        in_specs=[pl.BlockSpec((NB, E), lambda i: (i, 0))],
        out_specs=[
            pl.BlockSpec((NB, E), lambda i: (i, 0)),
            pl.BlockSpec((1, E), lambda i: (0, 0)),
        ],
        out_shape=[
            jax.ShapeDtypeStruct((N, E), jnp.float32),
            jax.ShapeDtypeStruct((1, E), jnp.float32),
        ],
        scratch_shapes=[pltpu.VMEM((1, E), jnp.float32)],
    )(topi)


def _slots_body(rank_ref, topi_ref, cnt_ref, cntc_ref, spos_ref, te_ref):
    cnt = cnt_ref[...]
    pad_cnt = jnp.floor((cnt + (T - 1)) * (1.0 / T)) * T
    e_r = lax.broadcasted_iota(jnp.int32, (E, E), 0)
    e_c = lax.broadcasted_iota(jnp.int32, (E, E), 1)
    strict_u = (e_r < e_c).astype(jnp.float32)
    base = lax.dot_general(pad_cnt, strict_u, (((1,), (0,)), ((), ())),
                           preferred_element_type=jnp.float32)
    ends = base + pad_cnt

    eio = lax.broadcasted_iota(jnp.int32, (NB, E), 1)
    oh0 = (topi_ref[:, 0:1] == eio).astype(jnp.float32)
    oh1 = (topi_ref[:, 1:2] == eio).astype(jnp.float32)
    s0 = rank_ref[:, 0:1] + jnp.sum(oh0 * base, axis=1, keepdims=True)
    s1 = rank_ref[:, 1:2] + jnp.sum(oh1 * base, axis=1, keepdims=True)
    spos = jnp.where(eio == 0, s0, jnp.where(eio == 1, s1, 0.0))
    spos_ref[...] = spos.astype(jnp.int32)

    tio = (lax.broadcasted_iota(jnp.int32, (NT, E), 0) * T).astype(jnp.float32)
    ge = (tio >= ends).astype(jnp.float32)
    te = jnp.minimum(jnp.sum(ge, axis=1), float(E - 1))
    te_row = te[None, :]

    k_r = lax.broadcasted_iota(jnp.int32, (NT, NT), 0)
    t_c = lax.broadcasted_iota(jnp.int32, (NT, NT), 1)
    shift = (k_r == t_c - 1).astype(jnp.float32)
    low_i = (k_r <= t_c).astype(jnp.float32)
    te_prev = lax.dot_general(te_row, shift, (((1,), (0,)), ((), ())),
                              preferred_element_type=jnp.float32)
    tlane = lax.broadcasted_iota(jnp.int32, (1, NT), 1)
    fr = jnp.where((te_row != te_prev) | (tlane == 0), 1.0, 0.0)
    rid = lax.dot_general(fr, low_i, (((1,), (0,)), ((), ())),
                          preferred_element_type=jnp.float32) - 1.0
    pr = rid - 2.0 * jnp.floor(rid * 0.5)

    cntc = cntc_ref[...]
    pad_cnt_c = jnp.floor((cntc + (T - 1)) * (1.0 / T)) * T
    e_col = lax.broadcasted_iota(jnp.int32, (E, 1), 0).astype(jnp.float32)
    used = jnp.sum(pad_cnt_c)
    present = (pad_cnt_c > 0.0) | ((e_col == E - 1) & (used < float(S)))
    cand = jnp.where(present & (e_col > te_row), e_col, float(E))
    nxe = jnp.min(cand, axis=0)[None, :]
    hn = jnp.where(nxe < float(E), 1.0, 0.0)
    nxe = jnp.minimum(nxe, float(E - 1))

    r8 = lax.broadcasted_iota(jnp.int32, (8, NT), 0)
    aux = jnp.where(r8 == 0, jnp.broadcast_to(te_row, (8, NT)),
          jnp.where(r8 == 1, jnp.broadcast_to(fr, (8, NT)),
          jnp.where(r8 == 2, jnp.broadcast_to(pr, (8, NT)),
          jnp.where(r8 == 3, jnp.broadcast_to(nxe, (8, NT)),
          jnp.where(r8 == 4, jnp.broadcast_to(hn, (8, NT)), 0.0)))))
    te_ref[...] = aux.astype(jnp.int32)


def _slots(rank, topi, cnt):
    return pl.pallas_call(
        _slots_body,
        grid=(N // NB,),
        in_specs=[
            pl.BlockSpec((NB, E), lambda i: (i, 0)),
            pl.BlockSpec((NB, E), lambda i: (i, 0)),
            pl.BlockSpec((1, E), lambda i: (0, 0)),
            pl.BlockSpec((E, 1), lambda i: (0, 0)),
        ],
        out_specs=[
            pl.BlockSpec((NB, E), lambda i: (i, 0)),
            pl.BlockSpec((8, NT), lambda i: (0, 0)),
        ],
        out_shape=[
            jax.ShapeDtypeStruct((N, E), jnp.int32),
            jax.ShapeDtypeStruct((8, NT), jnp.int32),
        ],
    )(rank, topi, cnt, cnt.reshape(E, 1))


def _dispatch_scatter_sc(x1, idx_flat):
    info = plsc.get_sparse_core_info()
    nw = info.num_cores * info.num_subcores
    bpw = N // nw
    mesh = plsc.VectorSubcoreMesh(core_axis_name="c", subcore_axis_name="s")

    @functools.partial(
        pl.kernel,
        out_type=jax.ShapeDtypeStruct((S, D), jnp.float32),
        mesh=mesh,
        scratch_types=[
            pltpu.VMEM((bpw, D), jnp.float32),
            pltpu.VMEM((bpw,), jnp.int32),
            pltpu.VMEM((bpw,), jnp.int32),
            pltpu.SemaphoreType.DMA,
            pltpu.SemaphoreType.DMA,
        ],
    )
    def k(x_hbm, idx_hbm, xs_hbm, rows_v, i0_v, i1_v, sem0, sem1):
        wid = lax.axis_index("s") * info.num_cores + lax.axis_index("c")
        base = wid * bpw
        pltpu.sync_copy(x_hbm.at[pl.ds(base, bpw)], rows_v)
        pltpu.sync_copy(idx_hbm.at[pl.ds(base, bpw)], i0_v)
        pltpu.sync_copy(idx_hbm.at[pl.ds(N + base, bpw)], i1_v)
        c0 = pltpu.async_copy(rows_v, xs_hbm.at[i0_v], sem0)
        c1 = pltpu.async_copy(rows_v, xs_hbm.at[i1_v], sem1)
        c0.wait()
        c1.wait()

    return k(x1, idx_flat)


def _combine_gather_sc(xout, idx_flat):
    info = plsc.get_sparse_core_info()
    nw = info.num_cores * info.num_subcores
    bpw = N // nw
    mesh = plsc.VectorSubcoreMesh(core_axis_name="c", subcore_axis_name="s")

    @functools.partial(
        pl.kernel,
        out_type=[
            jax.ShapeDtypeStruct((N, D), jnp.float32),
            jax.ShapeDtypeStruct((N, D), jnp.float32),
        ],
        mesh=mesh,
        scratch_types=[
            pltpu.VMEM((bpw, D), jnp.float32),
            pltpu.VMEM((bpw, D), jnp.float32),
            pltpu.VMEM((bpw,), jnp.int32),
            pltpu.VMEM((bpw,), jnp.int32),
            pltpu.SemaphoreType.DMA,
            pltpu.SemaphoreType.DMA,
        ],
    )
    def k(xo_hbm, idx_hbm, g0_hbm, g1_hbm, r0_v, r1_v, i0_v, i1_v, sem0, sem1):
        wid = lax.axis_index("s") * info.num_cores + lax.axis_index("c")
        base = wid * bpw
        pltpu.sync_copy(idx_hbm.at[pl.ds(base, bpw)], i0_v)
        pltpu.sync_copy(idx_hbm.at[pl.ds(N + base, bpw)], i1_v)
        c0 = pltpu.async_copy(xo_hbm.at[i0_v], r0_v, sem0)
        c1 = pltpu.async_copy(xo_hbm.at[i1_v], r1_v, sem1)
        c0.wait()
        c1.wait()
        pltpu.sync_copy(r0_v, g0_hbm.at[pl.ds(base, bpw)])
        pltpu.sync_copy(r1_v, g1_hbm.at[pl.ds(base, bpw)])

    return k(xout, idx_flat)


def _ffn_body(te_ref, fr_ref, pr_ref, nxe_ref, hn_ref,
              xs_ref, b1_ref, b2_ref, bo_ref, w1_hbm, w2_hbm, wo_hbm,
              out_ref, w1s_ref, w2s_ref, wos_ref,
              w1b_ref, w2b_ref, wob_ref, sem):
    j = pl.program_id(0)
    i = pl.program_id(1)

    def _issue(e_, slot):
        pltpu.make_async_copy(
            w1_hbm.at[e_, pl.ds(j * FB, FB), :], w1s_ref.at[slot],
            sem.at[0, slot]).start()
        pltpu.make_async_copy(
            w2_hbm.at[e_, pl.ds(j * FB, FB), :], w2s_ref.at[slot],
            sem.at[1, slot]).start()
        pltpu.make_async_copy(
            wo_hbm.at[e_, :, pl.ds(j * FB, FB)], wos_ref.at[slot],
            sem.at[2, slot]).start()

    def _wait(e_, slot):
        pltpu.make_async_copy(
            w1_hbm.at[e_, pl.ds(j * FB, FB), :], w1s_ref.at[slot],
            sem.at[0, slot]).wait()
        pltpu.make_async_copy(
            w2_hbm.at[e_, pl.ds(j * FB, FB), :], w2s_ref.at[slot],
            sem.at[1, slot]).wait()
        pltpu.make_async_copy(
            wo_hbm.at[e_, :, pl.ds(j * FB, FB)], wos_ref.at[slot],
            sem.at[2, slot]).wait()

    @pl.when(i == 0)
    def _():
        _issue(te_ref[0], 0)

    @pl.when(fr_ref[i] == 1)
    def _():
        slot = pr_ref[i]

        @pl.when(slot == 0)
        def _():
            _wait(te_ref[i], 0)
            w1b_ref[...] = w1s_ref[0].astype(jnp.bfloat16)
            w2b_ref[...] = w2s_ref[0].astype(jnp.bfloat16)
            wob_ref[...] = wos_ref[0].astype(jnp.bfloat16)

            @pl.when(hn_ref[i] == 1)
            def _():
                _issue(nxe_ref[i], 1)

        @pl.when(slot == 1)
        def _():
            _wait(te_ref[i], 1)
            w1b_ref[...] = w1s_ref[1].astype(jnp.bfloat16)
            w2b_ref[...] = w2s_ref[1].astype(jnp.bfloat16)
            wob_ref[...] = wos_ref[1].astype(jnp.bfloat16)

            @pl.when(hn_ref[i] == 1)
            def _():
                _issue(nxe_ref[i], 0)

    x = xs_ref[...].astype(jnp.bfloat16)
    h1 = lax.dot_general(x, w1b_ref[...], (((1,), (1,)), ((), ())),
                         preferred_element_type=jnp.float32) + b1_ref[0]
    h2 = lax.dot_general(x, w2b_ref[...], (((1,), (1,)), ((), ())),
                         preferred_element_type=jnp.float32) + b2_ref[0]
    g = (h1 / (1.0 + jnp.exp(-h1))) * h2
    g = g.astype(jnp.bfloat16)
    acc = lax.dot_general(g, wob_ref[...], (((1,), (1,)), ((), ())),
                          preferred_element_type=jnp.float32)

    @pl.when(j == 0)
    def _():
        out_ref[pl.ds(i * T, T), :] = acc + bo_ref[0]

    @pl.when(j > 0)
    def _():
        out_ref[pl.ds(i * T, T), :] += acc


def _grouped_ffn(te, fr, pr, nxe, hn, xs, W1, b1, W2, b2, Wout, bout):
    grid_spec = pltpu.PrefetchScalarGridSpec(
        num_scalar_prefetch=5,
        grid=(NJ, NT),
        in_specs=[
            pl.BlockSpec((T, D), lambda j, i, *_: (i, 0)),
            pl.BlockSpec((1, 1, FB),
                         lambda j, i, te, *_: (te[i] * NJ + j, 0, 0)),
            pl.BlockSpec((1, 1, FB),
                         lambda j, i, te, *_: (te[i] * NJ + j, 0, 0)),
            pl.BlockSpec((1, 1, D), lambda j, i, te, *_: (te[i], 0, 0)),
            pl.BlockSpec(memory_space=pltpu.MemorySpace.HBM),
            pl.BlockSpec(memory_space=pltpu.MemorySpace.HBM),
            pl.BlockSpec(memory_space=pltpu.MemorySpace.HBM),
        ],
        out_specs=pl.BlockSpec((S, D), lambda j, i, *_: (0, 0)),
        scratch_shapes=[
            pltpu.VMEM((2, FB, D), jnp.float32),
            pltpu.VMEM((2, FB, D), jnp.float32),
            pltpu.VMEM((2, D, FB), jnp.float32),
            pltpu.VMEM((FB, D), jnp.bfloat16),
            pltpu.VMEM((FB, D), jnp.bfloat16),
            pltpu.VMEM((D, FB), jnp.bfloat16),
            pltpu.SemaphoreType.DMA((3, 2)),
        ],
    )
    return pl.pallas_call(
        _ffn_body,
        grid_spec=grid_spec,
        out_shape=jax.ShapeDtypeStruct((S, D), jnp.float32),
    )(te, fr, pr, nxe, hn, xs, b1.reshape(E * NJ, 1, FB),
      b2.reshape(E * NJ, 1, FB), bout.reshape(E, 1, D), W1, W2, Wout)


def _final_body(x1_ref, g0_ref, g1_ref, tw_ref, n2w_ref, n2b_ref, o_ref):
    w0 = tw_ref[:, 0:1]
    w1 = tw_ref[:, 1:2]
    y = x1_ref[...] + w0 * g0_ref[...] + w1 * g1_ref[...]
    mu = jnp.mean(y, axis=1, keepdims=True)
    yc = y - mu
    var = jnp.mean(yc * yc, axis=1, keepdims=True)
    o_ref[...] = yc * lax.rsqrt(var + EPS) * n2w_ref[...] + n2b_ref[...]


def _final(x1, g0, g1, topw, norm2_w, norm2_b):
    return pl.pallas_call(
        _final_body,
        grid=(N // NB,),
        in_specs=[
            pl.BlockSpec((NB, D), lambda i: (i, 0)),
            pl.BlockSpec((NB, D), lambda i: (i, 0)),
            pl.BlockSpec((NB, D), lambda i: (i, 0)),
            pl.BlockSpec((NB, E), lambda i: (i, 0)),
            pl.BlockSpec((1, D), lambda i: (0, 0)),
            pl.BlockSpec((1, D), lambda i: (0, 0)),
        ],
        out_specs=pl.BlockSpec((NB, D), lambda i: (i, 0)),
        out_shape=jax.ShapeDtypeStruct((N, D), jnp.float32),
    )(x1, g0, g1, topw, norm2_w.reshape(1, D), norm2_b.reshape(1, D))


def kernel(src, in_proj_w, in_proj_b, out_proj_w, out_proj_b,
           norm1_w, norm1_b, norm2_w, norm2_b,
           gate_w, gate_b, W1, b1, W2, b2, Wout, bout):
    qkv = _qkv(src, in_proj_w, in_proj_b)
    qkvh = qkv.reshape(N, 3 * H, DH).transpose(1, 0, 2)
    oh = _attention(qkvh)
    o = oh.transpose(1, 0, 2).reshape(N, D)
    x1, topi, topw = _post_attn(o, src, out_proj_w, out_proj_b,
                                norm1_w, norm1_b, gate_w, gate_b)
    rank, cnt = _ranks(topi)
    spos, aux = _slots(rank, topi, cnt)
    idx_flat = jnp.concatenate([spos[:, 0], spos[:, 1]])
    xs = _dispatch_scatter_sc(x1, idx_flat)
    xout = _grouped_ffn(aux[0], aux[1], aux[2], aux[3], aux[4],
                        xs, W1, b1, W2, b2, Wout, bout)
    g0, g1 = _combine_gather_sc(xout, idx_flat)
    return _final(x1, g0, g1, topw, norm2_w, norm2_b)


# speedup vs baseline: 1.0788x; 1.0788x over previous
import functools

import jax
import jax.numpy as jnp
from jax import lax
from jax.experimental import pallas as pl
from jax.experimental.pallas import tpu as pltpu
from jax.experimental.pallas import tpu_sc as plsc

N = 2048
D = 768
H = 12
DH = D // H
FF = 3072
E = 8
K = 2
EPS = 1e-05

T = 256
S = N * K + E * T
NT = S // T
FB = 1024
NJ = FF // FB

NB = 256
QB = 1024


def _qkv_body(x_ref, w_ref, b_ref, o_ref, ws_ref):
    @pl.when(pl.program_id(0) == 0)
    def _():
        ws_ref[...] = w_ref[...].astype(jnp.bfloat16)

    x = x_ref[...].astype(jnp.bfloat16)
    acc = lax.dot_general(x, ws_ref[...], (((1,), (1,)), ((), ())),
                          preferred_element_type=jnp.float32)
    acc = acc + b_ref[...]
    cio = lax.broadcasted_iota(jnp.int32, (1, 3 * D), 1)
    acc = acc * jnp.where(cio < D, 1.0 / (DH ** 0.5), 1.0)
    o_ref[...] = acc.astype(jnp.bfloat16)


def _qkv(src, in_proj_w, in_proj_b):
    return pl.pallas_call(
        _qkv_body,
        grid=(N // NB,),
        in_specs=[
            pl.BlockSpec((NB, D), lambda i: (i, 0)),
            pl.BlockSpec((3 * D, D), lambda i: (0, 0)),
            pl.BlockSpec((1, 3 * D), lambda i: (0, 0)),
        ],
        out_specs=pl.BlockSpec((NB, 3 * D), lambda i: (i, 0)),
        out_shape=jax.ShapeDtypeStruct((N, 3 * D), jnp.bfloat16),
        scratch_shapes=[pltpu.VMEM((3 * D, D), jnp.bfloat16)],
    )(src, in_proj_w, in_proj_b.reshape(1, 3 * D))


def _attn_body(q_ref, k_ref, v_ref, o_ref):
    q = q_ref[0]
    k = k_ref[0]
    s = lax.dot_general(q, k, (((1,), (1,)), ((), ())),
                        preferred_element_type=jnp.float32)
    m = jnp.max(s, axis=1, keepdims=True)
    p = jnp.exp(s - m)
    l = jnp.sum(p, axis=1, keepdims=True)
    o = lax.dot_general(p.astype(jnp.bfloat16), v_ref[0],
                        (((1,), (0,)), ((), ())),
                        preferred_element_type=jnp.float32)
    o_ref[0] = (o * (1.0 / l)).astype(jnp.bfloat16)


def _attention(qkvh):
    return pl.pallas_call(
        _attn_body,
        grid=(H, N // QB),
        in_specs=[
            pl.BlockSpec((1, QB, DH), lambda h, i: (h, i, 0)),
            pl.BlockSpec((1, N, DH), lambda h, i: (H + h, 0, 0)),
            pl.BlockSpec((1, N, DH), lambda h, i: (2 * H + h, 0, 0)),
        ],
        out_specs=pl.BlockSpec((1, QB, DH), lambda h, i: (h, i, 0)),
        out_shape=jax.ShapeDtypeStruct((H, N, DH), jnp.bfloat16),
    )(qkvh, qkvh, qkvh)


def _post_attn_body(o_ref, src_ref, wo_ref, bo_ref, n1w_ref, n1b_ref,
                    gw_ref, gb_ref, x1_ref, topi_ref, topw_ref, ws_ref):
    @pl.when(pl.program_id(0) == 0)
    def _():
        ws_ref[...] = wo_ref[...].astype(jnp.bfloat16)

    o = o_ref[...]
    sa = lax.dot_general(o, ws_ref[...], (((1,), (1,)), ((), ())),
                         preferred_element_type=jnp.float32)
    y = sa + bo_ref[...] + src_ref[...]
    mu = jnp.mean(y, axis=1, keepdims=True)
    yc = y - mu
    var = jnp.mean(yc * yc, axis=1, keepdims=True)
    x1 = yc * lax.rsqrt(var + EPS) * n1w_ref[...] + n1b_ref[...]
    x1_ref[...] = x1

    logits = lax.dot_general(x1, gw_ref[...], (((1,), (1,)), ((), ())),
                             preferred_element_type=jnp.float32) + gb_ref[...]
    lm = jnp.max(logits, axis=1, keepdims=True)
    eg = jnp.exp(logits - lm)
    g = eg / jnp.sum(eg, axis=1, keepdims=True)
    eio = lax.broadcasted_iota(jnp.int32, (NB, E), 1)
    m1 = jnp.max(g, axis=1, keepdims=True)
    i1 = jnp.min(jnp.where(g == m1, eio, E), axis=1, keepdims=True)
    g2 = jnp.where(eio == i1, -1.0, g)
    m2 = jnp.max(g2, axis=1, keepdims=True)
    i2 = jnp.min(jnp.where(g2 == m2, eio, E), axis=1, keepdims=True)
    topi_ref[...] = jnp.where(eio == 0, i1, jnp.where(eio == 1, i2, 0))
    topw_ref[...] = jnp.where(eio == 0, m1, jnp.where(eio == 1, m2, 0.0))


def _post_attn(o, src, out_proj_w, out_proj_b, norm1_w, norm1_b, gate_w, gate_b):
    return pl.pallas_call(
        _post_attn_body,
        grid=(N // NB,),
        in_specs=[
            pl.BlockSpec((NB, D), lambda i: (i, 0)),
            pl.BlockSpec((NB, D), lambda i: (i, 0)),
            pl.BlockSpec((D, D), lambda i: (0, 0)),
            pl.BlockSpec((1, D), lambda i: (0, 0)),
            pl.BlockSpec((1, D), lambda i: (0, 0)),
            pl.BlockSpec((1, D), lambda i: (0, 0)),
            pl.BlockSpec((E, D), lambda i: (0, 0)),
            pl.BlockSpec((1, E), lambda i: (0, 0)),
        ],
        out_specs=[
            pl.BlockSpec((NB, D), lambda i: (i, 0)),
            pl.BlockSpec((NB, E), lambda i: (i, 0)),
            pl.BlockSpec((NB, E), lambda i: (i, 0)),
        ],
        out_shape=[
            jax.ShapeDtypeStruct((N, D), jnp.float32),
            jax.ShapeDtypeStruct((N, E), jnp.int32),
            jax.ShapeDtypeStruct((N, E), jnp.float32),
        ],
        scratch_shapes=[pltpu.VMEM((D, D), jnp.bfloat16)],
    )(o, src, out_proj_w, out_proj_b.reshape(1, D), norm1_w.reshape(1, D),
      norm1_b.reshape(1, D), gate_w, gate_b.reshape(1, E))


def _rank_body(topi_ref, rank_ref, cnt_ref, tot_ref):
    i = pl.program_id(0)

    @pl.when(i == 0)
    def _():
        tot_ref[...] = jnp.zeros_like(tot_ref)

    eio = lax.broadcasted_iota(jnp.int32, (NB, E), 1)
    oh0 = (topi_ref[:, 0:1] == eio).astype(jnp.float32)
    oh1 = (topi_ref[:, 1:2] == eio).astype(jnp.float32)
    c = oh0 + oh1
    r = lax.broadcasted_iota(jnp.int32, (NB, NB), 0)
    cc = lax.broadcasted_iota(jnp.int32, (NB, NB), 1)
    strict_l = (r > cc).astype(jnp.float32)
    pre = lax.dot_general(strict_l, c, (((1,), (0,)), ((), ())),
                          preferred_element_type=jnp.float32)
    base = tot_ref[...] + pre
    r0 = jnp.sum(oh0 * base, axis=1, keepdims=True)
    r1 = jnp.sum(oh1 * base, axis=1, keepdims=True)
    eiof = lax.broadcasted_iota(jnp.int32, (NB, E), 1)
    rank_ref[...] = jnp.where(eiof == 0, r0, jnp.where(eiof == 1, r1, 0.0))
    tot_ref[...] = tot_ref[0:1, :] + jnp.sum(c, axis=0, keepdims=True)
    cnt_ref[...] = tot_ref[0:1, :]


def _ranks(topi):
    return pl.pallas_call(
        _rank_body,
        grid=(N // NB,),
        in_specs=[pl.BlockSpec((NB, E), lambda i: (i, 0))],
        out_specs=[
            pl.BlockSpec((NB, E), lambda i: (i, 0)),
            pl.BlockSpec((1, E), lambda i: (0, 0)),
        ],
        out_shape=[
            jax.ShapeDtypeStruct((N, E), jnp.float32),
            jax.ShapeDtypeStruct((1, E), jnp.float32),
        ],
        scratch_shapes=[pltpu.VMEM((1, E), jnp.float32)],
    )(topi)


def _slots_body(rank_ref, topi_ref, cnt_ref, cntc_ref, spos_ref, te_ref):
    cnt = cnt_ref[...]
    pad_cnt = jnp.floor((cnt + (T - 1)) * (1.0 / T)) * T
    e_r = lax.broadcasted_iota(jnp.int32, (E, E), 0)
    e_c = lax.broadcasted_iota(jnp.int32, (E, E), 1)
    strict_u = (e_r < e_c).astype(jnp.float32)
    base = lax.dot_general(pad_cnt, strict_u, (((1,), (0,)), ((), ())),
                           preferred_element_type=jnp.float32)
    ends = base + pad_cnt

    eio = lax.broadcasted_iota(jnp.int32, (NB, E), 1)
    oh0 = (topi_ref[:, 0:1] == eio).astype(jnp.float32)
    oh1 = (topi_ref[:, 1:2] == eio).astype(jnp.float32)
    s0 = rank_ref[:, 0:1] + jnp.sum(oh0 * base, axis=1, keepdims=True)
    s1 = rank_ref[:, 1:2] + jnp.sum(oh1 * base, axis=1, keepdims=True)
    spos = jnp.where(eio == 0, s0, jnp.where(eio == 1, s1, 0.0))
    spos_ref[...] = spos.astype(jnp.int32)

    tio = (lax.broadcasted_iota(jnp.int32, (NT, E), 0) * T).astype(jnp.float32)
    ge = (tio >= ends).astype(jnp.float32)
    te = jnp.minimum(jnp.sum(ge, axis=1), float(E - 1))
    te_row = te[None, :]

    k_r = lax.broadcasted_iota(jnp.int32, (NT, NT), 0)
    t_c = lax.broadcasted_iota(jnp.int32, (NT, NT), 1)
    shift = (k_r == t_c - 1).astype(jnp.float32)
    low_i = (k_r <= t_c).astype(jnp.float32)
    te_prev = lax.dot_general(te_row, shift, (((1,), (0,)), ((), ())),
                              preferred_element_type=jnp.float32)
    tlane = lax.broadcasted_iota(jnp.int32, (1, NT), 1)
    fr = jnp.where((te_row != te_prev) | (tlane == 0), 1.0, 0.0)
    rid = lax.dot_general(fr, low_i, (((1,), (0,)), ((), ())),
                          preferred_element_type=jnp.float32) - 1.0
    pr = rid - 2.0 * jnp.floor(rid * 0.5)

    cntc = cntc_ref[...]
    pad_cnt_c = jnp.floor((cntc + (T - 1)) * (1.0 / T)) * T
    e_col = lax.broadcasted_iota(jnp.int32, (E, 1), 0).astype(jnp.float32)
    used = jnp.sum(pad_cnt_c)
    present = (pad_cnt_c > 0.0) | ((e_col == E - 1) & (used < float(S)))
    cand = jnp.where(present & (e_col > te_row), e_col, float(E))
    nxe = jnp.min(cand, axis=0)[None, :]
    hn = jnp.where(nxe < float(E), 1.0, 0.0)
    nxe = jnp.minimum(nxe, float(E - 1))

    r8 = lax.broadcasted_iota(jnp.int32, (8, NT), 0)
    aux = jnp.where(r8 == 0, jnp.broadcast_to(te_row, (8, NT)),
          jnp.where(r8 == 1, jnp.broadcast_to(fr, (8, NT)),
          jnp.where(r8 == 2, jnp.broadcast_to(pr, (8, NT)),
          jnp.where(r8 == 3, jnp.broadcast_to(nxe, (8, NT)),
          jnp.where(r8 == 4, jnp.broadcast_to(hn, (8, NT)), 0.0)))))
    te_ref[...] = aux.astype(jnp.int32)


def _slots(rank, topi, cnt):
    return pl.pallas_call(
        _slots_body,
        grid=(N // NB,),
        in_specs=[
            pl.BlockSpec((NB, E), lambda i: (i, 0)),
            pl.BlockSpec((NB, E), lambda i: (i, 0)),
            pl.BlockSpec((1, E), lambda i: (0, 0)),
            pl.BlockSpec((E, 1), lambda i: (0, 0)),
        ],
        out_specs=[
            pl.BlockSpec((NB, E), lambda i: (i, 0)),
            pl.BlockSpec((8, NT), lambda i: (0, 0)),
        ],
        out_shape=[
            jax.ShapeDtypeStruct((N, E), jnp.int32),
            jax.ShapeDtypeStruct((8, NT), jnp.int32),
        ],
    )(rank, topi, cnt, cnt.reshape(E, 1))


def _dispatch_scatter_sc(x1, idx_flat):
    info = plsc.get_sparse_core_info()
    nw = info.num_cores * info.num_subcores
    bpw = N // nw
    mesh = plsc.VectorSubcoreMesh(core_axis_name="c", subcore_axis_name="s")

    @functools.partial(
        pl.kernel,
        out_type=jax.ShapeDtypeStruct((S, D), jnp.float32),
        mesh=mesh,
        scratch_types=[
            pltpu.VMEM((bpw, D), jnp.float32),
            pltpu.VMEM((bpw,), jnp.int32),
            pltpu.VMEM((bpw,), jnp.int32),
            pltpu.SemaphoreType.DMA,
            pltpu.SemaphoreType.DMA,
        ],
    )
    def k(x_hbm, idx_hbm, xs_hbm, rows_v, i0_v, i1_v, sem0, sem1):
        wid = lax.axis_index("s") * info.num_cores + lax.axis_index("c")
        base = wid * bpw
        pltpu.sync_copy(x_hbm.at[pl.ds(base, bpw)], rows_v)
        pltpu.sync_copy(idx_hbm.at[pl.ds(base, bpw)], i0_v)
        pltpu.sync_copy(idx_hbm.at[pl.ds(N + base, bpw)], i1_v)
        c0 = pltpu.async_copy(rows_v, xs_hbm.at[i0_v], sem0)
        c1 = pltpu.async_copy(rows_v, xs_hbm.at[i1_v], sem1)
        c0.wait()
        c1.wait()

    return k(x1, idx_flat)


def _combine_gather_sc(xout, idx_flat):
    info = plsc.get_sparse_core_info()
    nw = info.num_cores * info.num_subcores
    bpw = N // nw
    mesh = plsc.VectorSubcoreMesh(core_axis_name="c", subcore_axis_name="s")

    @functools.partial(
        pl.kernel,
        out_type=[
            jax.ShapeDtypeStruct((N, D), jnp.float32),
            jax.ShapeDtypeStruct((N, D), jnp.float32),
        ],
        mesh=mesh,
        scratch_types=[
            pltpu.VMEM((bpw, D), jnp.float32),
            pltpu.VMEM((bpw, D), jnp.float32),
            pltpu.VMEM((bpw,), jnp.int32),
            pltpu.VMEM((bpw,), jnp.int32),
            pltpu.SemaphoreType.DMA,
            pltpu.SemaphoreType.DMA,
        ],
    )
    def k(xo_hbm, idx_hbm, g0_hbm, g1_hbm, r0_v, r1_v, i0_v, i1_v, sem0, sem1):
        wid = lax.axis_index("s") * info.num_cores + lax.axis_index("c")
        base = wid * bpw
        pltpu.sync_copy(idx_hbm.at[pl.ds(base, bpw)], i0_v)
        pltpu.sync_copy(idx_hbm.at[pl.ds(N + base, bpw)], i1_v)
        c0 = pltpu.async_copy(xo_hbm.at[i0_v], r0_v, sem0)
        c1 = pltpu.async_copy(xo_hbm.at[i1_v], r1_v, sem1)
        c0.wait()
        c1.wait()
        pltpu.sync_copy(r0_v, g0_hbm.at[pl.ds(base, bpw)])
        pltpu.sync_copy(r1_v, g1_hbm.at[pl.ds(base, bpw)])

    return k(xout, idx_flat)


def _ffn_body(te_ref, fr_ref, pr_ref, nxe_ref, hn_ref,
              xs_ref, b1_ref, b2_ref, bo_ref, w1_hbm, w2_hbm, wo_hbm,
              out_ref, w1s_ref, w2s_ref, wos_ref,
              w1b_ref, w2b_ref, wob_ref, sem):
    j = pl.program_id(0)
    i = pl.program_id(1)

    def _issue(e_, slot):
        pltpu.make_async_copy(
            w1_hbm.at[e_, pl.ds(j * FB, FB), :], w1s_ref.at[slot],
            sem.at[0, slot]).start()
        pltpu.make_async_copy(
            w2_hbm.at[e_, pl.ds(j * FB, FB), :], w2s_ref.at[slot],
            sem.at[1, slot]).start()
        pltpu.make_async_copy(
            wo_hbm.at[e_, :, pl.ds(j * FB, FB)], wos_ref.at[slot],
            sem.at[2, slot]).start()

    def _wait(e_, slot):
        pltpu.make_async_copy(
            w1_hbm.at[e_, pl.ds(j * FB, FB), :], w1s_ref.at[slot],
            sem.at[0, slot]).wait()
        pltpu.make_async_copy(
            w2_hbm.at[e_, pl.ds(j * FB, FB), :], w2s_ref.at[slot],
            sem.at[1, slot]).wait()
        pltpu.make_async_copy(
            wo_hbm.at[e_, :, pl.ds(j * FB, FB)], wos_ref.at[slot],
            sem.at[2, slot]).wait()

    @pl.when(i == 0)
    def _():
        _issue(te_ref[0], 0)

    @pl.when(fr_ref[i] == 1)
    def _():
        slot = pr_ref[i]

        @pl.when(slot == 0)
        def _():
            _wait(te_ref[i], 0)
            w1b_ref[...] = w1s_ref[0].astype(jnp.bfloat16)
            w2b_ref[...] = w2s_ref[0].astype(jnp.bfloat16)
            wob_ref[...] = wos_ref[0].astype(jnp.bfloat16)

            @pl.when(hn_ref[i] == 1)
            def _():
                _issue(nxe_ref[i], 1)

        @pl.when(slot == 1)
        def _():
            _wait(te_ref[i], 1)
            w1b_ref[...] = w1s_ref[1].astype(jnp.bfloat16)
            w2b_ref[...] = w2s_ref[1].astype(jnp.bfloat16)
            wob_ref[...] = wos_ref[1].astype(jnp.bfloat16)

            @pl.when(hn_ref[i] == 1)
            def _():
                _issue(nxe_ref[i], 0)

    x = xs_ref[...].astype(jnp.bfloat16)
    h1 = lax.dot_general(x, w1b_ref[...], (((1,), (1,)), ((), ())),
                         preferred_element_type=jnp.float32) + b1_ref[0]
    h2 = lax.dot_general(x, w2b_ref[...], (((1,), (1,)), ((), ())),
                         preferred_element_type=jnp.float32) + b2_ref[0]
    g = (h1 / (1.0 + jnp.exp(-h1))) * h2
    g = g.astype(jnp.bfloat16)
    acc = lax.dot_general(g, wob_ref[...], (((1,), (1,)), ((), ())),
                          preferred_element_type=jnp.float32)

    @pl.when(j == 0)
    def _():
        out_ref[pl.ds(i * T, T), :] = acc + bo_ref[0]

    @pl.when(j > 0)
    def _():
        out_ref[pl.ds(i * T, T), :] += acc


def _grouped_ffn(te, fr, pr, nxe, hn, xs, W1, b1, W2, b2, Wout, bout):
    grid_spec = pltpu.PrefetchScalarGridSpec(
        num_scalar_prefetch=5,
        grid=(NJ, NT),
        in_specs=[
            pl.BlockSpec((T, D), lambda j, i, *_: (i, 0)),
            pl.BlockSpec((1, 1, FB),
                         lambda j, i, te, *_: (te[i] * NJ + j, 0, 0)),
            pl.BlockSpec((1, 1, FB),
                         lambda j, i, te, *_: (te[i] * NJ + j, 0, 0)),
            pl.BlockSpec((1, 1, D), lambda j, i, te, *_: (te[i], 0, 0)),
            pl.BlockSpec(memory_space=pltpu.MemorySpace.HBM),
            pl.BlockSpec(memory_space=pltpu.MemorySpace.HBM),
            pl.BlockSpec(memory_space=pltpu.MemorySpace.HBM),
        ],
        out_specs=pl.BlockSpec((S, D), lambda j, i, *_: (0, 0)),
        scratch_shapes=[
            pltpu.VMEM((2, FB, D), jnp.float32),
            pltpu.VMEM((2, FB, D), jnp.float32),
            pltpu.VMEM((2, D, FB), jnp.float32),
            pltpu.VMEM((FB, D), jnp.bfloat16),
            pltpu.VMEM((FB, D), jnp.bfloat16),
            pltpu.VMEM((D, FB), jnp.bfloat16),
            pltpu.SemaphoreType.DMA((3, 2)),
        ],
    )
    return pl.pallas_call(
        _ffn_body,
        grid_spec=grid_spec,
        out_shape=jax.ShapeDtypeStruct((S, D), jnp.float32),
    )(te, fr, pr, nxe, hn, xs, b1.reshape(E * NJ, 1, FB),
      b2.reshape(E * NJ, 1, FB), bout.reshape(E, 1, D), W1, W2, Wout)


def _final_body(x1_ref, g0_ref, g1_ref, tw_ref, n2w_ref, n2b_ref, o_ref):
    w0 = tw_ref[:, 0:1]
    w1 = tw_ref[:, 1:2]
    y = x1_ref[...] + w0 * g0_ref[...] + w1 * g1_ref[...]
    mu = jnp.mean(y, axis=1, keepdims=True)
    yc = y - mu
    var = jnp.mean(yc * yc, axis=1, keepdims=True)
    o_ref[...] = yc * lax.rsqrt(var + EPS) * n2w_ref[...] + n2b_ref[...]


def _final(x1, g0, g1, topw, norm2_w, norm2_b):
    return pl.pallas_call(
        _final_body,
        grid=(N // NB,),
        in_specs=[
            pl.BlockSpec((NB, D), lambda i: (i, 0)),
            pl.BlockSpec((NB, D), lambda i: (i, 0)),
            pl.BlockSpec((NB, D), lambda i: (i, 0)),
            pl.BlockSpec((NB, E), lambda i: (i, 0)),
            pl.BlockSpec((1, D), lambda i: (0, 0)),
            pl.BlockSpec((1, D), lambda i: (0, 0)),
        ],
        out_specs=pl.BlockSpec((NB, D), lambda i: (i, 0)),
        out_shape=jax.ShapeDtypeStruct((N, D), jnp.float32),
    )(x1, g0, g1, topw, norm2_w.reshape(1, D), norm2_b.reshape(1, D))


def kernel(src, in_proj_w, in_proj_b, out_proj_w, out_proj_b,
           norm1_w, norm1_b, norm2_w, norm2_b,
           gate_w, gate_b, W1, b1, W2, b2, Wout, bout):
    qkv = _qkv(src, in_proj_w, in_proj_b)
    qkvh = qkv.reshape(N, 3 * H, DH).transpose(1, 0, 2)
    oh = _attention(qkvh)
    o = oh.transpose(1, 0, 2).reshape(N, D)
    x1, topi, topw = _post_attn(o, src, out_proj_w, out_proj_b,
                                norm1_w, norm1_b, gate_w, gate_b)
    rank, cnt = _ranks(topi)
    spos, aux = _slots(rank, topi, cnt)
    idx_flat = jnp.concatenate([spos[:, 0], spos[:, 1]])
    xs = _dispatch_scatter_sc(x1, idx_flat)
    xout = _grouped_ffn(aux[0], aux[1], aux[2], aux[3], aux[4],
                        xs, W1, b1, W2, b2, Wout, bout)
    g0, g1 = _combine_gather_sc(xout, idx_flat)
    return _final(x1, g0, g1, topw, norm2_w, norm2_b)


# speedup vs baseline: 1.1327x; 1.0500x over previous
import functools

import jax
import jax.numpy as jnp
from jax import lax
from jax.experimental import pallas as pl
from jax.experimental.pallas import tpu as pltpu
from jax.experimental.pallas import tpu_sc as plsc

N = 2048
D = 768
H = 12
DH = D // H
FF = 3072
E = 8
K = 2
EPS = 1e-05

T = 256
S = N * K + E * T
NT = S // T
FB = 1024
NJ = FF // FB

NB = 256
QB = 1024


def _qkv_body(x_ref, w_ref, b_ref, o_ref, ws_ref):
    @pl.when(pl.program_id(0) == 0)
    def _():
        ws_ref[...] = w_ref[...].astype(jnp.bfloat16)

    x = x_ref[...].astype(jnp.bfloat16)
    acc = lax.dot_general(x, ws_ref[...], (((1,), (1,)), ((), ())),
                          preferred_element_type=jnp.float32)
    acc = acc + b_ref[...]
    cio = lax.broadcasted_iota(jnp.int32, (1, 3 * D), 1)
    acc = acc * jnp.where(cio < D, 1.0 / (DH ** 0.5), 1.0)
    val = acc.astype(jnp.bfloat16).reshape(NB, 3 * H, DH)
    o_ref[...] = jnp.transpose(val, (1, 0, 2))


def _qkv(src, in_proj_w, in_proj_b):
    return pl.pallas_call(
        _qkv_body,
        grid=(N // NB,),
        in_specs=[
            pl.BlockSpec((NB, D), lambda i: (i, 0)),
            pl.BlockSpec((3 * D, D), lambda i: (0, 0)),
            pl.BlockSpec((1, 3 * D), lambda i: (0, 0)),
        ],
        out_specs=pl.BlockSpec((3 * H, NB, DH), lambda i: (0, i, 0)),
        out_shape=jax.ShapeDtypeStruct((3 * H, N, DH), jnp.bfloat16),
        scratch_shapes=[pltpu.VMEM((3 * D, D), jnp.bfloat16)],
    )(src, in_proj_w, in_proj_b.reshape(1, 3 * D))


def _attn_body(q_ref, k_ref, v_ref, o_ref):
    q = q_ref[0]
    k = k_ref[0]
    s = lax.dot_general(q, k, (((1,), (1,)), ((), ())),
                        preferred_element_type=jnp.float32)
    m = jnp.max(s, axis=1, keepdims=True)
    p = jnp.exp(s - m)
    l = jnp.sum(p, axis=1, keepdims=True)
    o = lax.dot_general(p.astype(jnp.bfloat16), v_ref[0],
                        (((1,), (0,)), ((), ())),
                        preferred_element_type=jnp.float32)
    o_ref[0] = (o * (1.0 / l)).astype(jnp.bfloat16)


def _attention(qkvh):
    return pl.pallas_call(
        _attn_body,
        grid=(H, N // QB),
        in_specs=[
            pl.BlockSpec((1, QB, DH), lambda h, i: (h, i, 0)),
            pl.BlockSpec((1, N, DH), lambda h, i: (H + h, 0, 0)),
            pl.BlockSpec((1, N, DH), lambda h, i: (2 * H + h, 0, 0)),
        ],
        out_specs=pl.BlockSpec((1, QB, DH), lambda h, i: (h, i, 0)),
        out_shape=jax.ShapeDtypeStruct((H, N, DH), jnp.bfloat16),
    )(qkvh, qkvh, qkvh)


def _post_attn_body(o_ref, src_ref, wo_ref, bo_ref, n1w_ref, n1b_ref,
                    gw_ref, gb_ref, x1_ref, topi_ref, topw_ref, ws_ref):
    @pl.when(pl.program_id(0) == 0)
    def _():
        ws_ref[...] = wo_ref[...].astype(jnp.bfloat16)

    o = jnp.transpose(o_ref[...], (1, 0, 2)).reshape(NB, D)
    sa = lax.dot_general(o, ws_ref[...], (((1,), (1,)), ((), ())),
                         preferred_element_type=jnp.float32)
    y = sa + bo_ref[...] + src_ref[...]
    mu = jnp.mean(y, axis=1, keepdims=True)
    yc = y - mu
    var = jnp.mean(yc * yc, axis=1, keepdims=True)
    x1 = yc * lax.rsqrt(var + EPS) * n1w_ref[...] + n1b_ref[...]
    x1_ref[...] = x1

    logits = lax.dot_general(x1, gw_ref[...], (((1,), (1,)), ((), ())),
                             preferred_element_type=jnp.float32) + gb_ref[...]
    lm = jnp.max(logits, axis=1, keepdims=True)
    eg = jnp.exp(logits - lm)
    g = eg / jnp.sum(eg, axis=1, keepdims=True)
    eio = lax.broadcasted_iota(jnp.int32, (NB, E), 1)
    m1 = jnp.max(g, axis=1, keepdims=True)
    i1 = jnp.min(jnp.where(g == m1, eio, E), axis=1, keepdims=True)
    g2 = jnp.where(eio == i1, -1.0, g)
    m2 = jnp.max(g2, axis=1, keepdims=True)
    i2 = jnp.min(jnp.where(g2 == m2, eio, E), axis=1, keepdims=True)
    topi_ref[...] = jnp.where(eio == 0, i1, jnp.where(eio == 1, i2, 0))
    topw_ref[...] = jnp.where(eio == 0, m1, jnp.where(eio == 1, m2, 0.0))


def _post_attn(o, src, out_proj_w, out_proj_b, norm1_w, norm1_b, gate_w, gate_b):
    return pl.pallas_call(
        _post_attn_body,
        grid=(N // NB,),
        in_specs=[
            pl.BlockSpec((H, NB, DH), lambda i: (0, i, 0)),
            pl.BlockSpec((NB, D), lambda i: (i, 0)),
            pl.BlockSpec((D, D), lambda i: (0, 0)),
            pl.BlockSpec((1, D), lambda i: (0, 0)),
            pl.BlockSpec((1, D), lambda i: (0, 0)),
            pl.BlockSpec((1, D), lambda i: (0, 0)),
            pl.BlockSpec((E, D), lambda i: (0, 0)),
            pl.BlockSpec((1, E), lambda i: (0, 0)),
        ],
        out_specs=[
            pl.BlockSpec((NB, D), lambda i: (i, 0)),
            pl.BlockSpec((NB, E), lambda i: (i, 0)),
            pl.BlockSpec((NB, E), lambda i: (i, 0)),
        ],
        out_shape=[
            jax.ShapeDtypeStruct((N, D), jnp.float32),
            jax.ShapeDtypeStruct((N, E), jnp.int32),
            jax.ShapeDtypeStruct((N, E), jnp.float32),
        ],
        scratch_shapes=[pltpu.VMEM((D, D), jnp.bfloat16)],
    )(o, src, out_proj_w, out_proj_b.reshape(1, D), norm1_w.reshape(1, D),
      norm1_b.reshape(1, D), gate_w, gate_b.reshape(1, E))


def _rank_body(topi_ref, rank_ref, cnt_ref, tot_ref):
    i = pl.program_id(0)

    @pl.when(i == 0)
    def _():
        tot_ref[...] = jnp.zeros_like(tot_ref)

    eio = lax.broadcasted_iota(jnp.int32, (NB, E), 1)
    oh0 = (topi_ref[:, 0:1] == eio).astype(jnp.float32)
    oh1 = (topi_ref[:, 1:2] == eio).astype(jnp.float32)
    c = oh0 + oh1
    r = lax.broadcasted_iota(jnp.int32, (NB, NB), 0)
    cc = lax.broadcasted_iota(jnp.int32, (NB, NB), 1)
    strict_l = (r > cc).astype(jnp.float32)
    pre = lax.dot_general(strict_l, c, (((1,), (0,)), ((), ())),
                          preferred_element_type=jnp.float32)
    base = tot_ref[...] + pre
    r0 = jnp.sum(oh0 * base, axis=1, keepdims=True)
    r1 = jnp.sum(oh1 * base, axis=1, keepdims=True)
    eiof = lax.broadcasted_iota(jnp.int32, (NB, E), 1)
    rank_ref[...] = jnp.where(eiof == 0, r0, jnp.where(eiof == 1, r1, 0.0))
    tot_ref[...] = tot_ref[0:1, :] + jnp.sum(c, axis=0, keepdims=True)
    cnt_ref[...] = tot_ref[0:1, :]


def _ranks(topi):
    return pl.pallas_call(
        _rank_body,
        grid=(N // NB,),
        in_specs=[pl.BlockSpec((NB, E), lambda i: (i, 0))],
        out_specs=[
            pl.BlockSpec((NB, E), lambda i: (i, 0)),
            pl.BlockSpec((1, E), lambda i: (0, 0)),
        ],
        out_shape=[
            jax.ShapeDtypeStruct((N, E), jnp.float32),
            jax.ShapeDtypeStruct((1, E), jnp.float32),
        ],
        scratch_shapes=[pltpu.VMEM((1, E), jnp.float32)],
    )(topi)


def _slots_body(rank_ref, topi_ref, cnt_ref, cntc_ref, spos_ref, te_ref):
    cnt = cnt_ref[...]
    pad_cnt = jnp.floor((cnt + (T - 1)) * (1.0 / T)) * T
    e_r = lax.broadcasted_iota(jnp.int32, (E, E), 0)
    e_c = lax.broadcasted_iota(jnp.int32, (E, E), 1)
    strict_u = (e_r < e_c).astype(jnp.float32)
    base = lax.dot_general(pad_cnt, strict_u, (((1,), (0,)), ((), ())),
                           preferred_element_type=jnp.float32)
    ends = base + pad_cnt

    eio = lax.broadcasted_iota(jnp.int32, (NB, E), 1)
    oh0 = (topi_ref[:, 0:1] == eio).astype(jnp.float32)
    oh1 = (topi_ref[:, 1:2] == eio).astype(jnp.float32)
    s0 = rank_ref[:, 0:1] + jnp.sum(oh0 * base, axis=1, keepdims=True)
    s1 = rank_ref[:, 1:2] + jnp.sum(oh1 * base, axis=1, keepdims=True)
    spos = jnp.where(eio == 0, s0, jnp.where(eio == 1, s1, 0.0))
    spos_ref[...] = spos.astype(jnp.int32)

    tio = (lax.broadcasted_iota(jnp.int32, (NT, E), 0) * T).astype(jnp.float32)
    ge = (tio >= ends).astype(jnp.float32)
    te = jnp.minimum(jnp.sum(ge, axis=1), float(E - 1))
    te_row = te[None, :]

    k_r = lax.broadcasted_iota(jnp.int32, (NT, NT), 0)
    t_c = lax.broadcasted_iota(jnp.int32, (NT, NT), 1)
    shift = (k_r == t_c - 1).astype(jnp.float32)
    low_i = (k_r <= t_c).astype(jnp.float32)
    te_prev = lax.dot_general(te_row, shift, (((1,), (0,)), ((), ())),
                              preferred_element_type=jnp.float32)
    tlane = lax.broadcasted_iota(jnp.int32, (1, NT), 1)
    fr = jnp.where((te_row != te_prev) | (tlane == 0), 1.0, 0.0)
    rid = lax.dot_general(fr, low_i, (((1,), (0,)), ((), ())),
                          preferred_element_type=jnp.float32) - 1.0
    pr = rid - 2.0 * jnp.floor(rid * 0.5)

    cntc = cntc_ref[...]
    pad_cnt_c = jnp.floor((cntc + (T - 1)) * (1.0 / T)) * T
    e_col = lax.broadcasted_iota(jnp.int32, (E, 1), 0).astype(jnp.float32)
    used = jnp.sum(pad_cnt_c)
    present = (pad_cnt_c > 0.0) | ((e_col == E - 1) & (used < float(S)))
    cand = jnp.where(present & (e_col > te_row), e_col, float(E))
    nxe = jnp.min(cand, axis=0)[None, :]
    hn = jnp.where(nxe < float(E), 1.0, 0.0)
    nxe = jnp.minimum(nxe, float(E - 1))

    r8 = lax.broadcasted_iota(jnp.int32, (8, NT), 0)
    aux = jnp.where(r8 == 0, jnp.broadcast_to(te_row, (8, NT)),
          jnp.where(r8 == 1, jnp.broadcast_to(fr, (8, NT)),
          jnp.where(r8 == 2, jnp.broadcast_to(pr, (8, NT)),
          jnp.where(r8 == 3, jnp.broadcast_to(nxe, (8, NT)),
          jnp.where(r8 == 4, jnp.broadcast_to(hn, (8, NT)), 0.0)))))
    te_ref[...] = aux.astype(jnp.int32)


def _slots(rank, topi, cnt):
    return pl.pallas_call(
        _slots_body,
        grid=(N // NB,),
        in_specs=[
            pl.BlockSpec((NB, E), lambda i: (i, 0)),
            pl.BlockSpec((NB, E), lambda i: (i, 0)),
            pl.BlockSpec((1, E), lambda i: (0, 0)),
            pl.BlockSpec((E, 1), lambda i: (0, 0)),
        ],
        out_specs=[
            pl.BlockSpec((NB, E), lambda i: (i, 0)),
            pl.BlockSpec((8, NT), lambda i: (0, 0)),
        ],
        out_shape=[
            jax.ShapeDtypeStruct((N, E), jnp.int32),
            jax.ShapeDtypeStruct((8, NT), jnp.int32),
        ],
    )(rank, topi, cnt, cnt.reshape(E, 1))


def _dispatch_scatter_sc(x1, idx_flat):
    info = plsc.get_sparse_core_info()
    nw = info.num_cores * info.num_subcores
    bpw = N // nw
    mesh = plsc.VectorSubcoreMesh(core_axis_name="c", subcore_axis_name="s")

    @functools.partial(
        pl.kernel,
        out_type=jax.ShapeDtypeStruct((S, D), jnp.float32),
        mesh=mesh,
        scratch_types=[
            pltpu.VMEM((bpw, D), jnp.float32),
            pltpu.VMEM((bpw,), jnp.int32),
            pltpu.VMEM((bpw,), jnp.int32),
            pltpu.SemaphoreType.DMA,
            pltpu.SemaphoreType.DMA,
        ],
    )
    def k(x_hbm, idx_hbm, xs_hbm, rows_v, i0_v, i1_v, sem0, sem1):
        wid = lax.axis_index("s") * info.num_cores + lax.axis_index("c")
        base = wid * bpw
        pltpu.sync_copy(x_hbm.at[pl.ds(base, bpw)], rows_v)
        pltpu.sync_copy(idx_hbm.at[pl.ds(base, bpw)], i0_v)
        pltpu.sync_copy(idx_hbm.at[pl.ds(N + base, bpw)], i1_v)
        c0 = pltpu.async_copy(rows_v, xs_hbm.at[i0_v], sem0)
        c1 = pltpu.async_copy(rows_v, xs_hbm.at[i1_v], sem1)
        c0.wait()
        c1.wait()

    return k(x1, idx_flat)


def _combine_gather_sc(xout, idx_flat):
    info = plsc.get_sparse_core_info()
    nw = info.num_cores * info.num_subcores
    bpw = N // nw
    mesh = plsc.VectorSubcoreMesh(core_axis_name="c", subcore_axis_name="s")

    @functools.partial(
        pl.kernel,
        out_type=[
            jax.ShapeDtypeStruct((N, D), jnp.float32),
            jax.ShapeDtypeStruct((N, D), jnp.float32),
        ],
        mesh=mesh,
        scratch_types=[
            pltpu.VMEM((bpw, D), jnp.float32),
            pltpu.VMEM((bpw, D), jnp.float32),
            pltpu.VMEM((bpw,), jnp.int32),
            pltpu.VMEM((bpw,), jnp.int32),
            pltpu.SemaphoreType.DMA,
            pltpu.SemaphoreType.DMA,
        ],
    )
    def k(xo_hbm, idx_hbm, g0_hbm, g1_hbm, r0_v, r1_v, i0_v, i1_v, sem0, sem1):
        wid = lax.axis_index("s") * info.num_cores + lax.axis_index("c")
        base = wid * bpw
        pltpu.sync_copy(idx_hbm.at[pl.ds(base, bpw)], i0_v)
        pltpu.sync_copy(idx_hbm.at[pl.ds(N + base, bpw)], i1_v)
        c0 = pltpu.async_copy(xo_hbm.at[i0_v], r0_v, sem0)
        c1 = pltpu.async_copy(xo_hbm.at[i1_v], r1_v, sem1)
        c0.wait()
        c1.wait()
        pltpu.sync_copy(r0_v, g0_hbm.at[pl.ds(base, bpw)])
        pltpu.sync_copy(r1_v, g1_hbm.at[pl.ds(base, bpw)])

    return k(xout, idx_flat)


def _ffn_body(te_ref, fr_ref, pr_ref, nxe_ref, hn_ref,
              xs_ref, b1_ref, b2_ref, bo_ref, w1_hbm, w2_hbm, wo_hbm,
              out_ref, w1s_ref, w2s_ref, wos_ref,
              w1b_ref, w2b_ref, wob_ref, sem):
    j = pl.program_id(0)
    i = pl.program_id(1)

    def _issue(e_, slot):
        pltpu.make_async_copy(
            w1_hbm.at[e_, pl.ds(j * FB, FB), :], w1s_ref.at[slot],
            sem.at[0, slot]).start()
        pltpu.make_async_copy(
            w2_hbm.at[e_, pl.ds(j * FB, FB), :], w2s_ref.at[slot],
            sem.at[1, slot]).start()
        pltpu.make_async_copy(
            wo_hbm.at[e_, :, pl.ds(j * FB, FB)], wos_ref.at[slot],
            sem.at[2, slot]).start()

    def _wait(e_, slot):
        pltpu.make_async_copy(
            w1_hbm.at[e_, pl.ds(j * FB, FB), :], w1s_ref.at[slot],
            sem.at[0, slot]).wait()
        pltpu.make_async_copy(
            w2_hbm.at[e_, pl.ds(j * FB, FB), :], w2s_ref.at[slot],
            sem.at[1, slot]).wait()
        pltpu.make_async_copy(
            wo_hbm.at[e_, :, pl.ds(j * FB, FB)], wos_ref.at[slot],
            sem.at[2, slot]).wait()

    @pl.when(i == 0)
    def _():
        _issue(te_ref[0], 0)

    @pl.when(fr_ref[i] == 1)
    def _():
        slot = pr_ref[i]

        @pl.when(slot == 0)
        def _():
            _wait(te_ref[i], 0)
            w1b_ref[...] = w1s_ref[0].astype(jnp.bfloat16)
            w2b_ref[...] = w2s_ref[0].astype(jnp.bfloat16)
            wob_ref[...] = wos_ref[0].astype(jnp.bfloat16)

            @pl.when(hn_ref[i] == 1)
            def _():
                _issue(nxe_ref[i], 1)

        @pl.when(slot == 1)
        def _():
            _wait(te_ref[i], 1)
            w1b_ref[...] = w1s_ref[1].astype(jnp.bfloat16)
            w2b_ref[...] = w2s_ref[1].astype(jnp.bfloat16)
            wob_ref[...] = wos_ref[1].astype(jnp.bfloat16)

            @pl.when(hn_ref[i] == 1)
            def _():
                _issue(nxe_ref[i], 0)

    x = xs_ref[...].astype(jnp.bfloat16)
    h1 = lax.dot_general(x, w1b_ref[...], (((1,), (1,)), ((), ())),
                         preferred_element_type=jnp.float32) + b1_ref[0]
    h2 = lax.dot_general(x, w2b_ref[...], (((1,), (1,)), ((), ())),
                         preferred_element_type=jnp.float32) + b2_ref[0]
    g = (h1 / (1.0 + jnp.exp(-h1))) * h2
    g = g.astype(jnp.bfloat16)
    acc = lax.dot_general(g, wob_ref[...], (((1,), (1,)), ((), ())),
                          preferred_element_type=jnp.float32)

    @pl.when(j == 0)
    def _():
        out_ref[pl.ds(i * T, T), :] = acc + bo_ref[0]

    @pl.when(j > 0)
    def _():
        out_ref[pl.ds(i * T, T), :] += acc


def _grouped_ffn(te, fr, pr, nxe, hn, xs, W1, b1, W2, b2, Wout, bout):
    grid_spec = pltpu.PrefetchScalarGridSpec(
        num_scalar_prefetch=5,
        grid=(NJ, NT),
        in_specs=[
            pl.BlockSpec((T, D), lambda j, i, *_: (i, 0)),
            pl.BlockSpec((1, 1, FB),
                         lambda j, i, te, *_: (te[i] * NJ + j, 0, 0)),
            pl.BlockSpec((1, 1, FB),
                         lambda j, i, te, *_: (te[i] * NJ + j, 0, 0)),
            pl.BlockSpec((1, 1, D), lambda j, i, te, *_: (te[i], 0, 0)),
            pl.BlockSpec(memory_space=pltpu.MemorySpace.HBM),
            pl.BlockSpec(memory_space=pltpu.MemorySpace.HBM),
            pl.BlockSpec(memory_space=pltpu.MemorySpace.HBM),
        ],
        out_specs=pl.BlockSpec((S, D), lambda j, i, *_: (0, 0)),
        scratch_shapes=[
            pltpu.VMEM((2, FB, D), jnp.float32),
            pltpu.VMEM((2, FB, D), jnp.float32),
            pltpu.VMEM((2, D, FB), jnp.float32),
            pltpu.VMEM((FB, D), jnp.bfloat16),
            pltpu.VMEM((FB, D), jnp.bfloat16),
            pltpu.VMEM((D, FB), jnp.bfloat16),
            pltpu.SemaphoreType.DMA((3, 2)),
        ],
    )
    return pl.pallas_call(
        _ffn_body,
        grid_spec=grid_spec,
        out_shape=jax.ShapeDtypeStruct((S, D), jnp.float32),
    )(te, fr, pr, nxe, hn, xs, b1.reshape(E * NJ, 1, FB),
      b2.reshape(E * NJ, 1, FB), bout.reshape(E, 1, D), W1, W2, Wout)


def _final_body(x1_ref, g0_ref, g1_ref, tw_ref, n2w_ref, n2b_ref, o_ref):
    w0 = tw_ref[:, 0:1]
    w1 = tw_ref[:, 1:2]
    y = x1_ref[...] + w0 * g0_ref[...] + w1 * g1_ref[...]
    mu = jnp.mean(y, axis=1, keepdims=True)
    yc = y - mu
    var = jnp.mean(yc * yc, axis=1, keepdims=True)
    o_ref[...] = yc * lax.rsqrt(var + EPS) * n2w_ref[...] + n2b_ref[...]


def _final(x1, g0, g1, topw, norm2_w, norm2_b):
    return pl.pallas_call(
        _final_body,
        grid=(N // NB,),
        in_specs=[
            pl.BlockSpec((NB, D), lambda i: (i, 0)),
            pl.BlockSpec((NB, D), lambda i: (i, 0)),
            pl.BlockSpec((NB, D), lambda i: (i, 0)),
            pl.BlockSpec((NB, E), lambda i: (i, 0)),
            pl.BlockSpec((1, D), lambda i: (0, 0)),
            pl.BlockSpec((1, D), lambda i: (0, 0)),
        ],
        out_specs=pl.BlockSpec((NB, D), lambda i: (i, 0)),
        out_shape=jax.ShapeDtypeStruct((N, D), jnp.float32),
    )(x1, g0, g1, topw, norm2_w.reshape(1, D), norm2_b.reshape(1, D))


def kernel(src, in_proj_w, in_proj_b, out_proj_w, out_proj_b,
           norm1_w, norm1_b, norm2_w, norm2_b,
           gate_w, gate_b, W1, b1, W2, b2, Wout, bout):
    qkvh = _qkv(src, in_proj_w, in_proj_b)
    oh = _attention(qkvh)
    x1, topi, topw = _post_attn(oh, src, out_proj_w, out_proj_b,
                                norm1_w, norm1_b, gate_w, gate_b)
    rank, cnt = _ranks(topi)
    spos, aux = _slots(rank, topi, cnt)
    idx_flat = jnp.concatenate([spos[:, 0], spos[:, 1]])
    xs = _dispatch_scatter_sc(x1, idx_flat)
    xout = _grouped_ffn(aux[0], aux[1], aux[2], aux[3], aux[4],
                        xs, W1, b1, W2, b2, Wout, bout)
    g0, g1 = _combine_gather_sc(xout, idx_flat)
    return _final(x1, g0, g1, topw, norm2_w, norm2_b)


# speedup vs baseline: 1.1431x; 1.0092x over previous
import functools

import jax
import jax.numpy as jnp
from jax import lax
from jax.experimental import pallas as pl
from jax.experimental.pallas import tpu as pltpu
from jax.experimental.pallas import tpu_sc as plsc

N = 2048
D = 768
H = 12
DH = D // H
FF = 3072
E = 8
K = 2
EPS = 1e-05

T = 256
S = N * K + E * T
NT = S // T
FB = 1024
NJ = FF // FB

NB = 256
QB = 2048


def _qkv_body(x_ref, w_ref, b_ref, o_ref, ws_ref):
    @pl.when(pl.program_id(0) == 0)
    def _():
        ws_ref[...] = w_ref[...].astype(jnp.bfloat16)

    x = x_ref[...].astype(jnp.bfloat16)
    acc = lax.dot_general(x, ws_ref[...], (((1,), (1,)), ((), ())),
                          preferred_element_type=jnp.float32)
    acc = acc + b_ref[...]
    cio = lax.broadcasted_iota(jnp.int32, (1, 3 * D), 1)
    acc = acc * jnp.where(cio < D, 1.0 / (DH ** 0.5), 1.0)
    val = acc.astype(jnp.bfloat16).reshape(NB, 3 * H, DH)
    o_ref[...] = jnp.transpose(val, (1, 0, 2))


def _qkv(src, in_proj_w, in_proj_b):
    return pl.pallas_call(
        _qkv_body,
        grid=(N // NB,),
        in_specs=[
            pl.BlockSpec((NB, D), lambda i: (i, 0)),
            pl.BlockSpec((3 * D, D), lambda i: (0, 0)),
            pl.BlockSpec((1, 3 * D), lambda i: (0, 0)),
        ],
        out_specs=pl.BlockSpec((3 * H, NB, DH), lambda i: (0, i, 0)),
        out_shape=jax.ShapeDtypeStruct((3 * H, N, DH), jnp.bfloat16),
        scratch_shapes=[pltpu.VMEM((3 * D, D), jnp.bfloat16)],
    )(src, in_proj_w, in_proj_b.reshape(1, 3 * D))


def _attn_body(q_ref, k_ref, v_ref, o_ref):
    q = q_ref[0]
    k = k_ref[0]
    s = lax.dot_general(q, k, (((1,), (1,)), ((), ())),
                        preferred_element_type=jnp.float32)
    m = jnp.max(s, axis=1, keepdims=True)
    p = jnp.exp(s - m)
    l = jnp.sum(p, axis=1, keepdims=True)
    o = lax.dot_general(p.astype(jnp.bfloat16), v_ref[0],
                        (((1,), (0,)), ((), ())),
                        preferred_element_type=jnp.float32)
    o_ref[0] = (o * (1.0 / l)).astype(jnp.bfloat16)


def _attention(qkvh):
    return pl.pallas_call(
        _attn_body,
        grid=(H, N // QB),
        in_specs=[
            pl.BlockSpec((1, QB, DH), lambda h, i: (h, i, 0)),
            pl.BlockSpec((1, N, DH), lambda h, i: (H + h, 0, 0)),
            pl.BlockSpec((1, N, DH), lambda h, i: (2 * H + h, 0, 0)),
        ],
        out_specs=pl.BlockSpec((1, QB, DH), lambda h, i: (h, i, 0)),
        out_shape=jax.ShapeDtypeStruct((H, N, DH), jnp.bfloat16),
    )(qkvh, qkvh, qkvh)


def _post_attn_body(o_ref, src_ref, wo_ref, bo_ref, n1w_ref, n1b_ref,
                    gw_ref, gb_ref, x1_ref, topi_ref, topw_ref, ws_ref):
    @pl.when(pl.program_id(0) == 0)
    def _():
        ws_ref[...] = wo_ref[...].astype(jnp.bfloat16)

    o = jnp.transpose(o_ref[...], (1, 0, 2)).reshape(NB, D)
    sa = lax.dot_general(o, ws_ref[...], (((1,), (1,)), ((), ())),
                         preferred_element_type=jnp.float32)
    y = sa + bo_ref[...] + src_ref[...]
    mu = jnp.mean(y, axis=1, keepdims=True)
    yc = y - mu
    var = jnp.mean(yc * yc, axis=1, keepdims=True)
    x1 = yc * lax.rsqrt(var + EPS) * n1w_ref[...] + n1b_ref[...]
    x1_ref[...] = x1

    logits = lax.dot_general(x1, gw_ref[...], (((1,), (1,)), ((), ())),
                             preferred_element_type=jnp.float32) + gb_ref[...]
    lm = jnp.max(logits, axis=1, keepdims=True)
    eg = jnp.exp(logits - lm)
    g = eg / jnp.sum(eg, axis=1, keepdims=True)
    eio = lax.broadcasted_iota(jnp.int32, (NB, E), 1)
    m1 = jnp.max(g, axis=1, keepdims=True)
    i1 = jnp.min(jnp.where(g == m1, eio, E), axis=1, keepdims=True)
    g2 = jnp.where(eio == i1, -1.0, g)
    m2 = jnp.max(g2, axis=1, keepdims=True)
    i2 = jnp.min(jnp.where(g2 == m2, eio, E), axis=1, keepdims=True)
    topi_ref[...] = jnp.where(eio == 0, i1, jnp.where(eio == 1, i2, 0))
    topw_ref[...] = jnp.where(eio == 0, m1, jnp.where(eio == 1, m2, 0.0))


def _post_attn(o, src, out_proj_w, out_proj_b, norm1_w, norm1_b, gate_w, gate_b):
    return pl.pallas_call(
        _post_attn_body,
        grid=(N // NB,),
        in_specs=[
            pl.BlockSpec((H, NB, DH), lambda i: (0, i, 0)),
            pl.BlockSpec((NB, D), lambda i: (i, 0)),
            pl.BlockSpec((D, D), lambda i: (0, 0)),
            pl.BlockSpec((1, D), lambda i: (0, 0)),
            pl.BlockSpec((1, D), lambda i: (0, 0)),
            pl.BlockSpec((1, D), lambda i: (0, 0)),
            pl.BlockSpec((E, D), lambda i: (0, 0)),
            pl.BlockSpec((1, E), lambda i: (0, 0)),
        ],
        out_specs=[
            pl.BlockSpec((NB, D), lambda i: (i, 0)),
            pl.BlockSpec((NB, E), lambda i: (i, 0)),
            pl.BlockSpec((NB, E), lambda i: (i, 0)),
        ],
        out_shape=[
            jax.ShapeDtypeStruct((N, D), jnp.float32),
            jax.ShapeDtypeStruct((N, E), jnp.int32),
            jax.ShapeDtypeStruct((N, E), jnp.float32),
        ],
        scratch_shapes=[pltpu.VMEM((D, D), jnp.bfloat16)],
    )(o, src, out_proj_w, out_proj_b.reshape(1, D), norm1_w.reshape(1, D),
      norm1_b.reshape(1, D), gate_w, gate_b.reshape(1, E))


def _rank_body(topi_ref, rank_ref, cnt_ref, tot_ref):
    i = pl.program_id(0)

    @pl.when(i == 0)
    def _():
        tot_ref[...] = jnp.zeros_like(tot_ref)

    eio = lax.broadcasted_iota(jnp.int32, (NB, E), 1)
    oh0 = (topi_ref[:, 0:1] == eio).astype(jnp.float32)
    oh1 = (topi_ref[:, 1:2] == eio).astype(jnp.float32)
    c = oh0 + oh1
    r = lax.broadcasted_iota(jnp.int32, (NB, NB), 0)
    cc = lax.broadcasted_iota(jnp.int32, (NB, NB), 1)
    strict_l = (r > cc).astype(jnp.float32)
    pre = lax.dot_general(strict_l, c, (((1,), (0,)), ((), ())),
                          preferred_element_type=jnp.float32)
    base = tot_ref[...] + pre
    r0 = jnp.sum(oh0 * base, axis=1, keepdims=True)
    r1 = jnp.sum(oh1 * base, axis=1, keepdims=True)
    eiof = lax.broadcasted_iota(jnp.int32, (NB, E), 1)
    rank_ref[...] = jnp.where(eiof == 0, r0, jnp.where(eiof == 1, r1, 0.0))
    tot_ref[...] = tot_ref[0:1, :] + jnp.sum(c, axis=0, keepdims=True)
    cnt_ref[...] = tot_ref[0:1, :]


def _ranks(topi):
    return pl.pallas_call(
        _rank_body,
        grid=(N // NB,),
        in_specs=[pl.BlockSpec((NB, E), lambda i: (i, 0))],
        out_specs=[
            pl.BlockSpec((NB, E), lambda i: (i, 0)),
            pl.BlockSpec((1, E), lambda i: (0, 0)),
        ],
        out_shape=[
            jax.ShapeDtypeStruct((N, E), jnp.float32),
            jax.ShapeDtypeStruct((1, E), jnp.float32),
        ],
        scratch_shapes=[pltpu.VMEM((1, E), jnp.float32)],
    )(topi)


def _slots_body(rank_ref, topi_ref, cnt_ref, cntc_ref, spos_ref, te_ref):
    cnt = cnt_ref[...]
    pad_cnt = jnp.floor((cnt + (T - 1)) * (1.0 / T)) * T
    e_r = lax.broadcasted_iota(jnp.int32, (E, E), 0)
    e_c = lax.broadcasted_iota(jnp.int32, (E, E), 1)
    strict_u = (e_r < e_c).astype(jnp.float32)
    base = lax.dot_general(pad_cnt, strict_u, (((1,), (0,)), ((), ())),
                           preferred_element_type=jnp.float32)
    ends = base + pad_cnt

    eio = lax.broadcasted_iota(jnp.int32, (NB, E), 1)
    oh0 = (topi_ref[:, 0:1] == eio).astype(jnp.float32)
    oh1 = (topi_ref[:, 1:2] == eio).astype(jnp.float32)
    s0 = rank_ref[:, 0:1] + jnp.sum(oh0 * base, axis=1, keepdims=True)
    s1 = rank_ref[:, 1:2] + jnp.sum(oh1 * base, axis=1, keepdims=True)
    spos = jnp.where(eio == 0, s0, jnp.where(eio == 1, s1, 0.0))
    spos_ref[...] = spos.astype(jnp.int32)

    tio = (lax.broadcasted_iota(jnp.int32, (NT, E), 0) * T).astype(jnp.float32)
    ge = (tio >= ends).astype(jnp.float32)
    te = jnp.minimum(jnp.sum(ge, axis=1), float(E - 1))
    te_row = te[None, :]

    k_r = lax.broadcasted_iota(jnp.int32, (NT, NT), 0)
    t_c = lax.broadcasted_iota(jnp.int32, (NT, NT), 1)
    shift = (k_r == t_c - 1).astype(jnp.float32)
    low_i = (k_r <= t_c).astype(jnp.float32)
    te_prev = lax.dot_general(te_row, shift, (((1,), (0,)), ((), ())),
                              preferred_element_type=jnp.float32)
    tlane = lax.broadcasted_iota(jnp.int32, (1, NT), 1)
    fr = jnp.where((te_row != te_prev) | (tlane == 0), 1.0, 0.0)
    rid = lax.dot_general(fr, low_i, (((1,), (0,)), ((), ())),
                          preferred_element_type=jnp.float32) - 1.0
    pr = rid - 2.0 * jnp.floor(rid * 0.5)

    cntc = cntc_ref[...]
    pad_cnt_c = jnp.floor((cntc + (T - 1)) * (1.0 / T)) * T
    e_col = lax.broadcasted_iota(jnp.int32, (E, 1), 0).astype(jnp.float32)
    used = jnp.sum(pad_cnt_c)
    present = (pad_cnt_c > 0.0) | ((e_col == E - 1) & (used < float(S)))
    cand = jnp.where(present & (e_col > te_row), e_col, float(E))
    nxe = jnp.min(cand, axis=0)[None, :]
    hn = jnp.where(nxe < float(E), 1.0, 0.0)
    nxe = jnp.minimum(nxe, float(E - 1))

    r8 = lax.broadcasted_iota(jnp.int32, (8, NT), 0)
    aux = jnp.where(r8 == 0, jnp.broadcast_to(te_row, (8, NT)),
          jnp.where(r8 == 1, jnp.broadcast_to(fr, (8, NT)),
          jnp.where(r8 == 2, jnp.broadcast_to(pr, (8, NT)),
          jnp.where(r8 == 3, jnp.broadcast_to(nxe, (8, NT)),
          jnp.where(r8 == 4, jnp.broadcast_to(hn, (8, NT)), 0.0)))))
    te_ref[...] = aux.astype(jnp.int32)


def _slots(rank, topi, cnt):
    return pl.pallas_call(
        _slots_body,
        grid=(N // NB,),
        in_specs=[
            pl.BlockSpec((NB, E), lambda i: (i, 0)),
            pl.BlockSpec((NB, E), lambda i: (i, 0)),
            pl.BlockSpec((1, E), lambda i: (0, 0)),
            pl.BlockSpec((E, 1), lambda i: (0, 0)),
        ],
        out_specs=[
            pl.BlockSpec((NB, E), lambda i: (i, 0)),
            pl.BlockSpec((8, NT), lambda i: (0, 0)),
        ],
        out_shape=[
            jax.ShapeDtypeStruct((N, E), jnp.int32),
            jax.ShapeDtypeStruct((8, NT), jnp.int32),
        ],
    )(rank, topi, cnt, cnt.reshape(E, 1))


def _dispatch_scatter_sc(x1, idx_flat):
    info = plsc.get_sparse_core_info()
    nw = info.num_cores * info.num_subcores
    bpw = N // nw
    mesh = plsc.VectorSubcoreMesh(core_axis_name="c", subcore_axis_name="s")

    @functools.partial(
        pl.kernel,
        out_type=jax.ShapeDtypeStruct((S, D), jnp.float32),
        mesh=mesh,
        scratch_types=[
            pltpu.VMEM((bpw, D), jnp.float32),
            pltpu.VMEM((bpw,), jnp.int32),
            pltpu.VMEM((bpw,), jnp.int32),
            pltpu.SemaphoreType.DMA,
            pltpu.SemaphoreType.DMA,
        ],
    )
    def k(x_hbm, idx_hbm, xs_hbm, rows_v, i0_v, i1_v, sem0, sem1):
        wid = lax.axis_index("s") * info.num_cores + lax.axis_index("c")
        base = wid * bpw
        pltpu.sync_copy(x_hbm.at[pl.ds(base, bpw)], rows_v)
        pltpu.sync_copy(idx_hbm.at[pl.ds(base, bpw)], i0_v)
        pltpu.sync_copy(idx_hbm.at[pl.ds(N + base, bpw)], i1_v)
        c0 = pltpu.async_copy(rows_v, xs_hbm.at[i0_v], sem0)
        c1 = pltpu.async_copy(rows_v, xs_hbm.at[i1_v], sem1)
        c0.wait()
        c1.wait()

    return k(x1, idx_flat)


def _combine_gather_sc(xout, idx_flat):
    info = plsc.get_sparse_core_info()
    nw = info.num_cores * info.num_subcores
    bpw = N // nw
    mesh = plsc.VectorSubcoreMesh(core_axis_name="c", subcore_axis_name="s")

    @functools.partial(
        pl.kernel,
        out_type=[
            jax.ShapeDtypeStruct((N, D), jnp.float32),
            jax.ShapeDtypeStruct((N, D), jnp.float32),
        ],
        mesh=mesh,
        scratch_types=[
            pltpu.VMEM((bpw, D), jnp.float32),
            pltpu.VMEM((bpw, D), jnp.float32),
            pltpu.VMEM((bpw,), jnp.int32),
            pltpu.VMEM((bpw,), jnp.int32),
            pltpu.SemaphoreType.DMA,
            pltpu.SemaphoreType.DMA,
        ],
    )
    def k(xo_hbm, idx_hbm, g0_hbm, g1_hbm, r0_v, r1_v, i0_v, i1_v, sem0, sem1):
        wid = lax.axis_index("s") * info.num_cores + lax.axis_index("c")
        base = wid * bpw
        pltpu.sync_copy(idx_hbm.at[pl.ds(base, bpw)], i0_v)
        pltpu.sync_copy(idx_hbm.at[pl.ds(N + base, bpw)], i1_v)
        c0 = pltpu.async_copy(xo_hbm.at[i0_v], r0_v, sem0)
        c1 = pltpu.async_copy(xo_hbm.at[i1_v], r1_v, sem1)
        c0.wait()
        c1.wait()
        pltpu.sync_copy(r0_v, g0_hbm.at[pl.ds(base, bpw)])
        pltpu.sync_copy(r1_v, g1_hbm.at[pl.ds(base, bpw)])

    return k(xout, idx_flat)


def _ffn_body(te_ref, fr_ref, pr_ref, nxe_ref, hn_ref,
              xs_ref, b1_ref, b2_ref, bo_ref, w1_hbm, w2_hbm, wo_hbm,
              out_ref, w1s_ref, w2s_ref, wos_ref,
              w1b_ref, w2b_ref, wob_ref, sem):
    j = pl.program_id(0)
    i = pl.program_id(1)

    def _issue(e_, slot):
        pltpu.make_async_copy(
            w1_hbm.at[e_, pl.ds(j * FB, FB), :], w1s_ref.at[slot],
            sem.at[0, slot]).start()
        pltpu.make_async_copy(
            w2_hbm.at[e_, pl.ds(j * FB, FB), :], w2s_ref.at[slot],
            sem.at[1, slot]).start()
        pltpu.make_async_copy(
            wo_hbm.at[e_, :, pl.ds(j * FB, FB)], wos_ref.at[slot],
            sem.at[2, slot]).start()

    def _wait(e_, slot):
        pltpu.make_async_copy(
            w1_hbm.at[e_, pl.ds(j * FB, FB), :], w1s_ref.at[slot],
            sem.at[0, slot]).wait()
        pltpu.make_async_copy(
            w2_hbm.at[e_, pl.ds(j * FB, FB), :], w2s_ref.at[slot],
            sem.at[1, slot]).wait()
        pltpu.make_async_copy(
            wo_hbm.at[e_, :, pl.ds(j * FB, FB)], wos_ref.at[slot],
            sem.at[2, slot]).wait()

    @pl.when(i == 0)
    def _():
        _issue(te_ref[0], 0)

    @pl.when(fr_ref[i] == 1)
    def _():
        slot = pr_ref[i]

        @pl.when(slot == 0)
        def _():
            _wait(te_ref[i], 0)
            w1b_ref[...] = w1s_ref[0].astype(jnp.bfloat16)
            w2b_ref[...] = w2s_ref[0].astype(jnp.bfloat16)
            wob_ref[...] = wos_ref[0].astype(jnp.bfloat16)

            @pl.when(hn_ref[i] == 1)
            def _():
                _issue(nxe_ref[i], 1)

        @pl.when(slot == 1)
        def _():
            _wait(te_ref[i], 1)
            w1b_ref[...] = w1s_ref[1].astype(jnp.bfloat16)
            w2b_ref[...] = w2s_ref[1].astype(jnp.bfloat16)
            wob_ref[...] = wos_ref[1].astype(jnp.bfloat16)

            @pl.when(hn_ref[i] == 1)
            def _():
                _issue(nxe_ref[i], 0)

    x = xs_ref[...].astype(jnp.bfloat16)
    h1 = lax.dot_general(x, w1b_ref[...], (((1,), (1,)), ((), ())),
                         preferred_element_type=jnp.float32) + b1_ref[0]
    h2 = lax.dot_general(x, w2b_ref[...], (((1,), (1,)), ((), ())),
                         preferred_element_type=jnp.float32) + b2_ref[0]
    g = (h1 / (1.0 + jnp.exp(-h1))) * h2
    g = g.astype(jnp.bfloat16)
    acc = lax.dot_general(g, wob_ref[...], (((1,), (1,)), ((), ())),
                          preferred_element_type=jnp.float32)

    @pl.when(j == 0)
    def _():
        out_ref[pl.ds(i * T, T), :] = acc + bo_ref[0]

    @pl.when(j > 0)
    def _():
        out_ref[pl.ds(i * T, T), :] += acc


def _grouped_ffn(te, fr, pr, nxe, hn, xs, W1, b1, W2, b2, Wout, bout):
    grid_spec = pltpu.PrefetchScalarGridSpec(
        num_scalar_prefetch=5,
        grid=(NJ, NT),
        in_specs=[
            pl.BlockSpec((T, D), lambda j, i, *_: (i, 0)),
            pl.BlockSpec((1, 1, FB),
                         lambda j, i, te, *_: (te[i] * NJ + j, 0, 0)),
            pl.BlockSpec((1, 1, FB),
                         lambda j, i, te, *_: (te[i] * NJ + j, 0, 0)),
            pl.BlockSpec((1, 1, D), lambda j, i, te, *_: (te[i], 0, 0)),
            pl.BlockSpec(memory_space=pltpu.MemorySpace.HBM),
            pl.BlockSpec(memory_space=pltpu.MemorySpace.HBM),
            pl.BlockSpec(memory_space=pltpu.MemorySpace.HBM),
        ],
        out_specs=pl.BlockSpec((S, D), lambda j, i, *_: (0, 0)),
        scratch_shapes=[
            pltpu.VMEM((2, FB, D), jnp.float32),
            pltpu.VMEM((2, FB, D), jnp.float32),
            pltpu.VMEM((2, D, FB), jnp.float32),
            pltpu.VMEM((FB, D), jnp.bfloat16),
            pltpu.VMEM((FB, D), jnp.bfloat16),
            pltpu.VMEM((D, FB), jnp.bfloat16),
            pltpu.SemaphoreType.DMA((3, 2)),
        ],
    )
    return pl.pallas_call(
        _ffn_body,
        grid_spec=grid_spec,
        out_shape=jax.ShapeDtypeStruct((S, D), jnp.float32),
    )(te, fr, pr, nxe, hn, xs, b1.reshape(E * NJ, 1, FB),
      b2.reshape(E * NJ, 1, FB), bout.reshape(E, 1, D), W1, W2, Wout)


def _final_body(x1_ref, g0_ref, g1_ref, tw_ref, n2w_ref, n2b_ref, o_ref):
    w0 = tw_ref[:, 0:1]
    w1 = tw_ref[:, 1:2]
    y = x1_ref[...] + w0 * g0_ref[...] + w1 * g1_ref[...]
    mu = jnp.mean(y, axis=1, keepdims=True)
    yc = y - mu
    var = jnp.mean(yc * yc, axis=1, keepdims=True)
    o_ref[...] = yc * lax.rsqrt(var + EPS) * n2w_ref[...] + n2b_ref[...]


def _final(x1, g0, g1, topw, norm2_w, norm2_b):
    return pl.pallas_call(
        _final_body,
        grid=(N // NB,),
        in_specs=[
            pl.BlockSpec((NB, D), lambda i: (i, 0)),
            pl.BlockSpec((NB, D), lambda i: (i, 0)),
            pl.BlockSpec((NB, D), lambda i: (i, 0)),
            pl.BlockSpec((NB, E), lambda i: (i, 0)),
            pl.BlockSpec((1, D), lambda i: (0, 0)),
            pl.BlockSpec((1, D), lambda i: (0, 0)),
        ],
        out_specs=pl.BlockSpec((NB, D), lambda i: (i, 0)),
        out_shape=jax.ShapeDtypeStruct((N, D), jnp.float32),
    )(x1, g0, g1, topw, norm2_w.reshape(1, D), norm2_b.reshape(1, D))


def kernel(src, in_proj_w, in_proj_b, out_proj_w, out_proj_b,
           norm1_w, norm1_b, norm2_w, norm2_b,
           gate_w, gate_b, W1, b1, W2, b2, Wout, bout):
    qkvh = _qkv(src, in_proj_w, in_proj_b)
    oh = _attention(qkvh)
    x1, topi, topw = _post_attn(oh, src, out_proj_w, out_proj_b,
                                norm1_w, norm1_b, gate_w, gate_b)
    rank, cnt = _ranks(topi)
    spos, aux = _slots(rank, topi, cnt)
    idx_flat = jnp.concatenate([spos[:, 0], spos[:, 1]])
    xs = _dispatch_scatter_sc(x1, idx_flat)
    xout = _grouped_ffn(aux[0], aux[1], aux[2], aux[3], aux[4],
                        xs, W1, b1, W2, b2, Wout, bout)
    g0, g1 = _combine_gather_sc(xout, idx_flat)
    return _final(x1, g0, g1, topw, norm2_w, norm2_b)


# speedup vs baseline: 1.1967x; 1.0469x over previous
import functools

import jax
import jax.numpy as jnp
from jax import lax
from jax.experimental import pallas as pl
from jax.experimental.pallas import tpu as pltpu
from jax.experimental.pallas import tpu_sc as plsc

N = 2048
D = 768
H = 12
DH = D // H
FF = 3072
E = 8
K = 2
EPS = 1e-05

T = 256
S = N * K + E * T
NT = S // T
FB = 1024
NJ = FF // FB

NB = 256
QB = 2048


def _qkv_body(x_ref, w_ref, b_ref, o_ref, ws_ref):
    @pl.when(pl.program_id(0) == 0)
    def _():
        ws_ref[...] = w_ref[...].astype(jnp.bfloat16)

    x = x_ref[...].astype(jnp.bfloat16)
    acc = lax.dot_general(ws_ref[...], x, (((1,), (1,)), ((), ())),
                          preferred_element_type=jnp.float32)
    acc = acc + b_ref[...]
    rio = lax.broadcasted_iota(jnp.int32, (3 * D, 1), 0)
    acc = acc * jnp.where(rio < D, 1.0 / (DH ** 0.5), 1.0)
    o_ref[...] = acc.astype(jnp.bfloat16).reshape(3 * H, DH, NB)


def _qkv(src, in_proj_w, in_proj_b):
    return pl.pallas_call(
        _qkv_body,
        grid=(N // NB,),
        in_specs=[
            pl.BlockSpec((NB, D), lambda i: (i, 0)),
            pl.BlockSpec((3 * D, D), lambda i: (0, 0)),
            pl.BlockSpec((3 * D, 1), lambda i: (0, 0)),
        ],
        out_specs=pl.BlockSpec((3 * H, DH, NB), lambda i: (0, 0, i)),
        out_shape=jax.ShapeDtypeStruct((3 * H, DH, N), jnp.bfloat16),
        scratch_shapes=[pltpu.VMEM((3 * D, D), jnp.bfloat16)],
    )(src, in_proj_w, in_proj_b.reshape(3 * D, 1))


def _attn_body(q_ref, k_ref, v_ref, o_ref):
    qt = q_ref[0]
    kt = k_ref[0]
    s = lax.dot_general(qt, kt, (((0,), (0,)), ((), ())),
                        preferred_element_type=jnp.float32)
    m = jnp.max(s, axis=1, keepdims=True)
    p = jnp.exp(s - m)
    l = jnp.sum(p, axis=1, keepdims=True)
    o = lax.dot_general(p.astype(jnp.bfloat16), v_ref[0],
                        (((1,), (1,)), ((), ())),
                        preferred_element_type=jnp.float32)
    o_ref[0] = (o * (1.0 / l)).astype(jnp.bfloat16)


def _attention(qkvt):
    return pl.pallas_call(
        _attn_body,
        grid=(H, N // QB),
        in_specs=[
            pl.BlockSpec((1, DH, QB), lambda h, i: (h, 0, i)),
            pl.BlockSpec((1, DH, N), lambda h, i: (H + h, 0, 0)),
            pl.BlockSpec((1, DH, N), lambda h, i: (2 * H + h, 0, 0)),
        ],
        out_specs=pl.BlockSpec((1, QB, DH), lambda h, i: (h, i, 0)),
        out_shape=jax.ShapeDtypeStruct((H, N, DH), jnp.bfloat16),
    )(qkvt, qkvt, qkvt)


def _post_attn_body(o_ref, src_ref, wo_ref, bo_ref, n1w_ref, n1b_ref,
                    gw_ref, gb_ref, x1_ref, topi_ref, topw_ref, ws_ref):
    @pl.when(pl.program_id(0) == 0)
    def _():
        ws_ref[...] = wo_ref[...].astype(jnp.bfloat16)

    o = jnp.transpose(o_ref[...], (1, 0, 2)).reshape(NB, D)
    sa = lax.dot_general(o, ws_ref[...], (((1,), (1,)), ((), ())),
                         preferred_element_type=jnp.float32)
    y = sa + bo_ref[...] + src_ref[...]
    mu = jnp.mean(y, axis=1, keepdims=True)
    yc = y - mu
    var = jnp.mean(yc * yc, axis=1, keepdims=True)
    x1 = yc * lax.rsqrt(var + EPS) * n1w_ref[...] + n1b_ref[...]
    x1_ref[...] = x1

    logits = lax.dot_general(x1, gw_ref[...], (((1,), (1,)), ((), ())),
                             preferred_element_type=jnp.float32) + gb_ref[...]
    lm = jnp.max(logits, axis=1, keepdims=True)
    eg = jnp.exp(logits - lm)
    g = eg / jnp.sum(eg, axis=1, keepdims=True)
    eio = lax.broadcasted_iota(jnp.int32, (NB, E), 1)
    m1 = jnp.max(g, axis=1, keepdims=True)
    i1 = jnp.min(jnp.where(g == m1, eio, E), axis=1, keepdims=True)
    g2 = jnp.where(eio == i1, -1.0, g)
    m2 = jnp.max(g2, axis=1, keepdims=True)
    i2 = jnp.min(jnp.where(g2 == m2, eio, E), axis=1, keepdims=True)
    topi_ref[...] = jnp.where(eio == 0, i1, jnp.where(eio == 1, i2, 0))
    topw_ref[...] = jnp.where(eio == 0, m1, jnp.where(eio == 1, m2, 0.0))


def _post_attn(o, src, out_proj_w, out_proj_b, norm1_w, norm1_b, gate_w, gate_b):
    return pl.pallas_call(
        _post_attn_body,
        grid=(N // NB,),
        in_specs=[
            pl.BlockSpec((H, NB, DH), lambda i: (0, i, 0)),
            pl.BlockSpec((NB, D), lambda i: (i, 0)),
            pl.BlockSpec((D, D), lambda i: (0, 0)),
            pl.BlockSpec((1, D), lambda i: (0, 0)),
            pl.BlockSpec((1, D), lambda i: (0, 0)),
            pl.BlockSpec((1, D), lambda i: (0, 0)),
            pl.BlockSpec((E, D), lambda i: (0, 0)),
            pl.BlockSpec((1, E), lambda i: (0, 0)),
        ],
        out_specs=[
            pl.BlockSpec((NB, D), lambda i: (i, 0)),
            pl.BlockSpec((NB, E), lambda i: (i, 0)),
            pl.BlockSpec((NB, E), lambda i: (i, 0)),
        ],
        out_shape=[
            jax.ShapeDtypeStruct((N, D), jnp.float32),
            jax.ShapeDtypeStruct((N, E), jnp.int32),
            jax.ShapeDtypeStruct((N, E), jnp.float32),
        ],
        scratch_shapes=[pltpu.VMEM((D, D), jnp.bfloat16)],
    )(o, src, out_proj_w, out_proj_b.reshape(1, D), norm1_w.reshape(1, D),
      norm1_b.reshape(1, D), gate_w, gate_b.reshape(1, E))


def _rank_body(topi_ref, rank_ref, cnt_ref, tot_ref):
    i = pl.program_id(0)

    @pl.when(i == 0)
    def _():
        tot_ref[...] = jnp.zeros_like(tot_ref)

    eio = lax.broadcasted_iota(jnp.int32, (NB, E), 1)
    oh0 = (topi_ref[:, 0:1] == eio).astype(jnp.float32)
    oh1 = (topi_ref[:, 1:2] == eio).astype(jnp.float32)
    c = oh0 + oh1
    r = lax.broadcasted_iota(jnp.int32, (NB, NB), 0)
    cc = lax.broadcasted_iota(jnp.int32, (NB, NB), 1)
    strict_l = (r > cc).astype(jnp.float32)
    pre = lax.dot_general(strict_l, c, (((1,), (0,)), ((), ())),
                          preferred_element_type=jnp.float32)
    base = tot_ref[...] + pre
    r0 = jnp.sum(oh0 * base, axis=1, keepdims=True)
    r1 = jnp.sum(oh1 * base, axis=1, keepdims=True)
    eiof = lax.broadcasted_iota(jnp.int32, (NB, E), 1)
    rank_ref[...] = jnp.where(eiof == 0, r0, jnp.where(eiof == 1, r1, 0.0))
    tot_ref[...] = tot_ref[0:1, :] + jnp.sum(c, axis=0, keepdims=True)
    cnt_ref[...] = tot_ref[0:1, :]


def _ranks(topi):
    return pl.pallas_call(
        _rank_body,
        grid=(N // NB,),
        in_specs=[pl.BlockSpec((NB, E), lambda i: (i, 0))],
        out_specs=[
            pl.BlockSpec((NB, E), lambda i: (i, 0)),
            pl.BlockSpec((1, E), lambda i: (0, 0)),
        ],
        out_shape=[
            jax.ShapeDtypeStruct((N, E), jnp.float32),
            jax.ShapeDtypeStruct((1, E), jnp.float32),
        ],
        scratch_shapes=[pltpu.VMEM((1, E), jnp.float32)],
    )(topi)


def _slots_body(rank_ref, topi_ref, cnt_ref, cntc_ref, spos_ref, te_ref):
    cnt = cnt_ref[...]
    pad_cnt = jnp.floor((cnt + (T - 1)) * (1.0 / T)) * T
    e_r = lax.broadcasted_iota(jnp.int32, (E, E), 0)
    e_c = lax.broadcasted_iota(jnp.int32, (E, E), 1)
    strict_u = (e_r < e_c).astype(jnp.float32)
    base = lax.dot_general(pad_cnt, strict_u, (((1,), (0,)), ((), ())),
                           preferred_element_type=jnp.float32)
    ends = base + pad_cnt

    eio = lax.broadcasted_iota(jnp.int32, (NB, E), 1)
    oh0 = (topi_ref[:, 0:1] == eio).astype(jnp.float32)
    oh1 = (topi_ref[:, 1:2] == eio).astype(jnp.float32)
    s0 = rank_ref[:, 0:1] + jnp.sum(oh0 * base, axis=1, keepdims=True)
    s1 = rank_ref[:, 1:2] + jnp.sum(oh1 * base, axis=1, keepdims=True)
    spos = jnp.where(eio == 0, s0, jnp.where(eio == 1, s1, 0.0))
    spos_ref[...] = spos.astype(jnp.int32)

    tio = (lax.broadcasted_iota(jnp.int32, (NT, E), 0) * T).astype(jnp.float32)
    ge = (tio >= ends).astype(jnp.float32)
    te = jnp.minimum(jnp.sum(ge, axis=1), float(E - 1))
    te_row = te[None, :]

    k_r = lax.broadcasted_iota(jnp.int32, (NT, NT), 0)
    t_c = lax.broadcasted_iota(jnp.int32, (NT, NT), 1)
    shift = (k_r == t_c - 1).astype(jnp.float32)
    low_i = (k_r <= t_c).astype(jnp.float32)
    te_prev = lax.dot_general(te_row, shift, (((1,), (0,)), ((), ())),
                              preferred_element_type=jnp.float32)
    tlane = lax.broadcasted_iota(jnp.int32, (1, NT), 1)
    fr = jnp.where((te_row != te_prev) | (tlane == 0), 1.0, 0.0)
    rid = lax.dot_general(fr, low_i, (((1,), (0,)), ((), ())),
                          preferred_element_type=jnp.float32) - 1.0
    pr = rid - 2.0 * jnp.floor(rid * 0.5)

    cntc = cntc_ref[...]
    pad_cnt_c = jnp.floor((cntc + (T - 1)) * (1.0 / T)) * T
    e_col = lax.broadcasted_iota(jnp.int32, (E, 1), 0).astype(jnp.float32)
    used = jnp.sum(pad_cnt_c)
    present = (pad_cnt_c > 0.0) | ((e_col == E - 1) & (used < float(S)))
    cand = jnp.where(present & (e_col > te_row), e_col, float(E))
    nxe = jnp.min(cand, axis=0)[None, :]
    hn = jnp.where(nxe < float(E), 1.0, 0.0)
    nxe = jnp.minimum(nxe, float(E - 1))

    r8 = lax.broadcasted_iota(jnp.int32, (8, NT), 0)
    aux = jnp.where(r8 == 0, jnp.broadcast_to(te_row, (8, NT)),
          jnp.where(r8 == 1, jnp.broadcast_to(fr, (8, NT)),
          jnp.where(r8 == 2, jnp.broadcast_to(pr, (8, NT)),
          jnp.where(r8 == 3, jnp.broadcast_to(nxe, (8, NT)),
          jnp.where(r8 == 4, jnp.broadcast_to(hn, (8, NT)), 0.0)))))
    te_ref[...] = aux.astype(jnp.int32)


def _slots(rank, topi, cnt):
    return pl.pallas_call(
        _slots_body,
        grid=(N // NB,),
        in_specs=[
            pl.BlockSpec((NB, E), lambda i: (i, 0)),
            pl.BlockSpec((NB, E), lambda i: (i, 0)),
            pl.BlockSpec((1, E), lambda i: (0, 0)),
            pl.BlockSpec((E, 1), lambda i: (0, 0)),
        ],
        out_specs=[
            pl.BlockSpec((NB, E), lambda i: (i, 0)),
            pl.BlockSpec((8, NT), lambda i: (0, 0)),
        ],
        out_shape=[
            jax.ShapeDtypeStruct((N, E), jnp.int32),
            jax.ShapeDtypeStruct((8, NT), jnp.int32),
        ],
    )(rank, topi, cnt, cnt.reshape(E, 1))


def _dispatch_scatter_sc(x1, idx_flat):
    info = plsc.get_sparse_core_info()
    nw = info.num_cores * info.num_subcores
    bpw = N // nw
    mesh = plsc.VectorSubcoreMesh(core_axis_name="c", subcore_axis_name="s")

    @functools.partial(
        pl.kernel,
        out_type=jax.ShapeDtypeStruct((S, D), jnp.float32),
        mesh=mesh,
        scratch_types=[
            pltpu.VMEM((bpw, D), jnp.float32),
            pltpu.VMEM((bpw,), jnp.int32),
            pltpu.VMEM((bpw,), jnp.int32),
            pltpu.SemaphoreType.DMA,
            pltpu.SemaphoreType.DMA,
        ],
    )
    def k(x_hbm, idx_hbm, xs_hbm, rows_v, i0_v, i1_v, sem0, sem1):
        wid = lax.axis_index("s") * info.num_cores + lax.axis_index("c")
        base = wid * bpw
        pltpu.sync_copy(x_hbm.at[pl.ds(base, bpw)], rows_v)
        pltpu.sync_copy(idx_hbm.at[pl.ds(base, bpw)], i0_v)
        pltpu.sync_copy(idx_hbm.at[pl.ds(N + base, bpw)], i1_v)
        c0 = pltpu.async_copy(rows_v, xs_hbm.at[i0_v], sem0)
        c1 = pltpu.async_copy(rows_v, xs_hbm.at[i1_v], sem1)
        c0.wait()
        c1.wait()

    return k(x1, idx_flat)


def _combine_gather_sc(xout, idx_flat):
    info = plsc.get_sparse_core_info()
    nw = info.num_cores * info.num_subcores
    bpw = N // nw
    mesh = plsc.VectorSubcoreMesh(core_axis_name="c", subcore_axis_name="s")

    @functools.partial(
        pl.kernel,
        out_type=[
            jax.ShapeDtypeStruct((N, D), jnp.float32),
            jax.ShapeDtypeStruct((N, D), jnp.float32),
        ],
        mesh=mesh,
        scratch_types=[
            pltpu.VMEM((bpw, D), jnp.float32),
            pltpu.VMEM((bpw, D), jnp.float32),
            pltpu.VMEM((bpw,), jnp.int32),
            pltpu.VMEM((bpw,), jnp.int32),
            pltpu.SemaphoreType.DMA,
            pltpu.SemaphoreType.DMA,
        ],
    )
    def k(xo_hbm, idx_hbm, g0_hbm, g1_hbm, r0_v, r1_v, i0_v, i1_v, sem0, sem1):
        wid = lax.axis_index("s") * info.num_cores + lax.axis_index("c")
        base = wid * bpw
        pltpu.sync_copy(idx_hbm.at[pl.ds(base, bpw)], i0_v)
        pltpu.sync_copy(idx_hbm.at[pl.ds(N + base, bpw)], i1_v)
        c0 = pltpu.async_copy(xo_hbm.at[i0_v], r0_v, sem0)
        c1 = pltpu.async_copy(xo_hbm.at[i1_v], r1_v, sem1)
        c0.wait()
        c1.wait()
        pltpu.sync_copy(r0_v, g0_hbm.at[pl.ds(base, bpw)])
        pltpu.sync_copy(r1_v, g1_hbm.at[pl.ds(base, bpw)])

    return k(xout, idx_flat)


def _ffn_body(te_ref, fr_ref, pr_ref, nxe_ref, hn_ref,
              xs_ref, b1_ref, b2_ref, bo_ref, w1_hbm, w2_hbm, wo_hbm,
              out_ref, w1s_ref, w2s_ref, wos_ref,
              w1b_ref, w2b_ref, wob_ref, sem):
    j = pl.program_id(0)
    i = pl.program_id(1)

    def _issue(e_, slot):
        pltpu.make_async_copy(
            w1_hbm.at[e_, pl.ds(j * FB, FB), :], w1s_ref.at[slot],
            sem.at[0, slot]).start()
        pltpu.make_async_copy(
            w2_hbm.at[e_, pl.ds(j * FB, FB), :], w2s_ref.at[slot],
            sem.at[1, slot]).start()
        pltpu.make_async_copy(
            wo_hbm.at[e_, :, pl.ds(j * FB, FB)], wos_ref.at[slot],
            sem.at[2, slot]).start()

    def _wait(e_, slot):
        pltpu.make_async_copy(
            w1_hbm.at[e_, pl.ds(j * FB, FB), :], w1s_ref.at[slot],
            sem.at[0, slot]).wait()
        pltpu.make_async_copy(
            w2_hbm.at[e_, pl.ds(j * FB, FB), :], w2s_ref.at[slot],
            sem.at[1, slot]).wait()
        pltpu.make_async_copy(
            wo_hbm.at[e_, :, pl.ds(j * FB, FB)], wos_ref.at[slot],
            sem.at[2, slot]).wait()

    @pl.when(i == 0)
    def _():
        _issue(te_ref[0], 0)

    @pl.when(fr_ref[i] == 1)
    def _():
        slot = pr_ref[i]

        @pl.when(slot == 0)
        def _():
            _wait(te_ref[i], 0)
            w1b_ref[...] = w1s_ref[0].astype(jnp.bfloat16)
            w2b_ref[...] = w2s_ref[0].astype(jnp.bfloat16)
            wob_ref[...] = wos_ref[0].astype(jnp.bfloat16)

            @pl.when(hn_ref[i] == 1)
            def _():
                _issue(nxe_ref[i], 1)

        @pl.when(slot == 1)
        def _():
            _wait(te_ref[i], 1)
            w1b_ref[...] = w1s_ref[1].astype(jnp.bfloat16)
            w2b_ref[...] = w2s_ref[1].astype(jnp.bfloat16)
            wob_ref[...] = wos_ref[1].astype(jnp.bfloat16)

            @pl.when(hn_ref[i] == 1)
            def _():
                _issue(nxe_ref[i], 0)

    x = xs_ref[...].astype(jnp.bfloat16)
    h1 = lax.dot_general(x, w1b_ref[...], (((1,), (1,)), ((), ())),
                         preferred_element_type=jnp.float32) + b1_ref[0]
    h2 = lax.dot_general(x, w2b_ref[...], (((1,), (1,)), ((), ())),
                         preferred_element_type=jnp.float32) + b2_ref[0]
    g = (h1 / (1.0 + jnp.exp(-h1))) * h2
    g = g.astype(jnp.bfloat16)
    acc = lax.dot_general(g, wob_ref[...], (((1,), (1,)), ((), ())),
                          preferred_element_type=jnp.float32)

    @pl.when(j == 0)
    def _():
        out_ref[pl.ds(i * T, T), :] = acc + bo_ref[0]

    @pl.when(j > 0)
    def _():
        out_ref[pl.ds(i * T, T), :] += acc


def _grouped_ffn(te, fr, pr, nxe, hn, xs, W1, b1, W2, b2, Wout, bout):
    grid_spec = pltpu.PrefetchScalarGridSpec(
        num_scalar_prefetch=5,
        grid=(NJ, NT),
        in_specs=[
            pl.BlockSpec((T, D), lambda j, i, *_: (i, 0)),
            pl.BlockSpec((1, 1, FB),
                         lambda j, i, te, *_: (te[i] * NJ + j, 0, 0)),
            pl.BlockSpec((1, 1, FB),
                         lambda j, i, te, *_: (te[i] * NJ + j, 0, 0)),
            pl.BlockSpec((1, 1, D), lambda j, i, te, *_: (te[i], 0, 0)),
            pl.BlockSpec(memory_space=pltpu.MemorySpace.HBM),
            pl.BlockSpec(memory_space=pltpu.MemorySpace.HBM),
            pl.BlockSpec(memory_space=pltpu.MemorySpace.HBM),
        ],
        out_specs=pl.BlockSpec((S, D), lambda j, i, *_: (0, 0)),
        scratch_shapes=[
            pltpu.VMEM((2, FB, D), jnp.float32),
            pltpu.VMEM((2, FB, D), jnp.float32),
            pltpu.VMEM((2, D, FB), jnp.float32),
            pltpu.VMEM((FB, D), jnp.bfloat16),
            pltpu.VMEM((FB, D), jnp.bfloat16),
            pltpu.VMEM((D, FB), jnp.bfloat16),
            pltpu.SemaphoreType.DMA((3, 2)),
        ],
    )
    return pl.pallas_call(
        _ffn_body,
        grid_spec=grid_spec,
        out_shape=jax.ShapeDtypeStruct((S, D), jnp.float32),
    )(te, fr, pr, nxe, hn, xs, b1.reshape(E * NJ, 1, FB),
      b2.reshape(E * NJ, 1, FB), bout.reshape(E, 1, D), W1, W2, Wout)


def _final_body(x1_ref, g0_ref, g1_ref, tw_ref, n2w_ref, n2b_ref, o_ref):
    w0 = tw_ref[:, 0:1]
    w1 = tw_ref[:, 1:2]
    y = x1_ref[...] + w0 * g0_ref[...] + w1 * g1_ref[...]
    mu = jnp.mean(y, axis=1, keepdims=True)
    yc = y - mu
    var = jnp.mean(yc * yc, axis=1, keepdims=True)
    o_ref[...] = yc * lax.rsqrt(var + EPS) * n2w_ref[...] + n2b_ref[...]


def _final(x1, g0, g1, topw, norm2_w, norm2_b):
    return pl.pallas_call(
        _final_body,
        grid=(N // NB,),
        in_specs=[
            pl.BlockSpec((NB, D), lambda i: (i, 0)),
            pl.BlockSpec((NB, D), lambda i: (i, 0)),
            pl.BlockSpec((NB, D), lambda i: (i, 0)),
            pl.BlockSpec((NB, E), lambda i: (i, 0)),
            pl.BlockSpec((1, D), lambda i: (0, 0)),
            pl.BlockSpec((1, D), lambda i: (0, 0)),
        ],
        out_specs=pl.BlockSpec((NB, D), lambda i: (i, 0)),
        out_shape=jax.ShapeDtypeStruct((N, D), jnp.float32),
    )(x1, g0, g1, topw, norm2_w.reshape(1, D), norm2_b.reshape(1, D))


def kernel(src, in_proj_w, in_proj_b, out_proj_w, out_proj_b,
           norm1_w, norm1_b, norm2_w, norm2_b,
           gate_w, gate_b, W1, b1, W2, b2, Wout, bout):
    qkvh = _qkv(src, in_proj_w, in_proj_b)
    oh = _attention(qkvh)
    x1, topi, topw = _post_attn(oh, src, out_proj_w, out_proj_b,
                                norm1_w, norm1_b, gate_w, gate_b)
    rank, cnt = _ranks(topi)
    spos, aux = _slots(rank, topi, cnt)
    idx_flat = jnp.concatenate([spos[:, 0], spos[:, 1]])
    xs = _dispatch_scatter_sc(x1, idx_flat)
    xout = _grouped_ffn(aux[0], aux[1], aux[2], aux[3], aux[4],
                        xs, W1, b1, W2, b2, Wout, bout)
    g0, g1 = _combine_gather_sc(xout, idx_flat)
    return _final(x1, g0, g1, topw, norm2_w, norm2_b)


# speedup vs baseline: 1.2013x; 1.0039x over previous
import functools

import jax
import jax.numpy as jnp
from jax import lax
from jax.experimental import pallas as pl
from jax.experimental.pallas import tpu as pltpu
from jax.experimental.pallas import tpu_sc as plsc

N = 2048
D = 768
H = 12
DH = D // H
FF = 3072
E = 8
K = 2
EPS = 1e-05

T = 256
S = N * K + E * T
NT = S // T
FB = 1024
NJ = FF // FB

NB = 256
QB = 2048


def _qkv_body(x_ref, w_ref, b_ref, o_ref, ws_ref):
    @pl.when(pl.program_id(0) == 0)
    def _():
        ws_ref[...] = w_ref[...].astype(jnp.bfloat16)

    x = x_ref[...].astype(jnp.bfloat16)
    acc = lax.dot_general(ws_ref[...], x, (((1,), (1,)), ((), ())),
                          preferred_element_type=jnp.float32)
    acc = acc + b_ref[...]
    rio = lax.broadcasted_iota(jnp.int32, (3 * D, 1), 0)
    acc = acc * jnp.where(rio < D, 1.0 / (DH ** 0.5), 1.0)
    o_ref[...] = acc.astype(jnp.bfloat16).reshape(3 * H, DH, NB)


def _qkv(src, in_proj_w, in_proj_b):
    return pl.pallas_call(
        _qkv_body,
        grid=(N // NB,),
        in_specs=[
            pl.BlockSpec((NB, D), lambda i: (i, 0)),
            pl.BlockSpec((3 * D, D), lambda i: (0, 0)),
            pl.BlockSpec((3 * D, 1), lambda i: (0, 0)),
        ],
        out_specs=pl.BlockSpec((3 * H, DH, NB), lambda i: (0, 0, i)),
        out_shape=jax.ShapeDtypeStruct((3 * H, DH, N), jnp.bfloat16),
        scratch_shapes=[pltpu.VMEM((3 * D, D), jnp.bfloat16)],
    )(src, in_proj_w, in_proj_b.reshape(3 * D, 1))


def _attn_body(q_ref, k_ref, v_ref, o_ref):
    qt = q_ref[0]
    kt = k_ref[0]
    s = lax.dot_general(qt, kt, (((0,), (0,)), ((), ())),
                        preferred_element_type=jnp.float32)
    m = jnp.max(s, axis=1, keepdims=True)
    p = jnp.exp(s - m)
    l = jnp.sum(p, axis=1, keepdims=True)
    o = lax.dot_general(p.astype(jnp.bfloat16), v_ref[0],
                        (((1,), (1,)), ((), ())),
                        preferred_element_type=jnp.float32)
    o_ref[0] = (o * (1.0 / l)).astype(jnp.bfloat16)


def _attention(qkvt):
    return pl.pallas_call(
        _attn_body,
        grid=(H, N // QB),
        in_specs=[
            pl.BlockSpec((1, DH, QB), lambda h, i: (h, 0, i)),
            pl.BlockSpec((1, DH, N), lambda h, i: (H + h, 0, 0)),
            pl.BlockSpec((1, DH, N), lambda h, i: (2 * H + h, 0, 0)),
        ],
        out_specs=pl.BlockSpec((1, QB, DH), lambda h, i: (h, i, 0)),
        out_shape=jax.ShapeDtypeStruct((H, N, DH), jnp.bfloat16),
    )(qkvt, qkvt, qkvt)


def _post_attn_body(o_ref, src_ref, wo_ref, bo_ref, n1w_ref, n1b_ref,
                    gw_ref, gb_ref, x1_ref, topi_ref, topw_ref, ws_ref):
    @pl.when(pl.program_id(0) == 0)
    def _():
        ws_ref[...] = wo_ref[...].astype(jnp.bfloat16)

    o = jnp.transpose(o_ref[...], (1, 0, 2)).reshape(NB, D)
    sa = lax.dot_general(o, ws_ref[...], (((1,), (1,)), ((), ())),
                         preferred_element_type=jnp.float32)
    y = sa + bo_ref[...] + src_ref[...]
    mu = jnp.mean(y, axis=1, keepdims=True)
    yc = y - mu
    var = jnp.mean(yc * yc, axis=1, keepdims=True)
    x1 = yc * lax.rsqrt(var + EPS) * n1w_ref[...] + n1b_ref[...]
    x1_ref[...] = x1

    logits = lax.dot_general(x1, gw_ref[...], (((1,), (1,)), ((), ())),
                             preferred_element_type=jnp.float32) + gb_ref[...]
    lm = jnp.max(logits, axis=1, keepdims=True)
    eg = jnp.exp(logits - lm)
    g = eg / jnp.sum(eg, axis=1, keepdims=True)
    eio = lax.broadcasted_iota(jnp.int32, (NB, E), 1)
    m1 = jnp.max(g, axis=1, keepdims=True)
    i1 = jnp.min(jnp.where(g == m1, eio, E), axis=1, keepdims=True)
    g2 = jnp.where(eio == i1, -1.0, g)
    m2 = jnp.max(g2, axis=1, keepdims=True)
    i2 = jnp.min(jnp.where(g2 == m2, eio, E), axis=1, keepdims=True)
    topi_ref[...] = jnp.where(eio == 0, i1, jnp.where(eio == 1, i2, 0))
    topw_ref[...] = jnp.where(eio == 0, m1, jnp.where(eio == 1, m2, 0.0))


def _post_attn(o, src, out_proj_w, out_proj_b, norm1_w, norm1_b, gate_w, gate_b):
    return pl.pallas_call(
        _post_attn_body,
        grid=(N // NB,),
        in_specs=[
            pl.BlockSpec((H, NB, DH), lambda i: (0, i, 0)),
            pl.BlockSpec((NB, D), lambda i: (i, 0)),
            pl.BlockSpec((D, D), lambda i: (0, 0)),
            pl.BlockSpec((1, D), lambda i: (0, 0)),
            pl.BlockSpec((1, D), lambda i: (0, 0)),
            pl.BlockSpec((1, D), lambda i: (0, 0)),
            pl.BlockSpec((E, D), lambda i: (0, 0)),
            pl.BlockSpec((1, E), lambda i: (0, 0)),
        ],
        out_specs=[
            pl.BlockSpec((NB, D), lambda i: (i, 0)),
            pl.BlockSpec((NB, E), lambda i: (i, 0)),
            pl.BlockSpec((NB, E), lambda i: (i, 0)),
        ],
        out_shape=[
            jax.ShapeDtypeStruct((N, D), jnp.float32),
            jax.ShapeDtypeStruct((N, E), jnp.int32),
            jax.ShapeDtypeStruct((N, E), jnp.float32),
        ],
        scratch_shapes=[pltpu.VMEM((D, D), jnp.bfloat16)],
    )(o, src, out_proj_w, out_proj_b.reshape(1, D), norm1_w.reshape(1, D),
      norm1_b.reshape(1, D), gate_w, gate_b.reshape(1, E))


def _rank_body(topi_ref, rank_ref, cnt_ref, tot_ref):
    i = pl.program_id(0)

    @pl.when(i == 0)
    def _():
        tot_ref[...] = jnp.zeros_like(tot_ref)

    eio = lax.broadcasted_iota(jnp.int32, (NB, E), 1)
    oh0 = (topi_ref[:, 0:1] == eio).astype(jnp.float32)
    oh1 = (topi_ref[:, 1:2] == eio).astype(jnp.float32)
    c = oh0 + oh1
    r = lax.broadcasted_iota(jnp.int32, (NB, NB), 0)
    cc = lax.broadcasted_iota(jnp.int32, (NB, NB), 1)
    strict_l = (r > cc).astype(jnp.float32)
    pre = lax.dot_general(strict_l, c, (((1,), (0,)), ((), ())),
                          preferred_element_type=jnp.float32)
    base = tot_ref[...] + pre
    r0 = jnp.sum(oh0 * base, axis=1, keepdims=True)
    r1 = jnp.sum(oh1 * base, axis=1, keepdims=True)
    eiof = lax.broadcasted_iota(jnp.int32, (NB, E), 1)
    rank_ref[...] = jnp.where(eiof == 0, r0, jnp.where(eiof == 1, r1, 0.0))
    tot_ref[...] = tot_ref[0:1, :] + jnp.sum(c, axis=0, keepdims=True)
    cnt_ref[...] = tot_ref[0:1, :]


def _ranks(topi):
    return pl.pallas_call(
        _rank_body,
        grid=(N // NB,),
        in_specs=[pl.BlockSpec((NB, E), lambda i: (i, 0))],
        out_specs=[
            pl.BlockSpec((NB, E), lambda i: (i, 0)),
            pl.BlockSpec((1, E), lambda i: (0, 0)),
        ],
        out_shape=[
            jax.ShapeDtypeStruct((N, E), jnp.float32),
            jax.ShapeDtypeStruct((1, E), jnp.float32),
        ],
        scratch_shapes=[pltpu.VMEM((1, E), jnp.float32)],
    )(topi)


def _slots_body(rank_ref, topi_ref, cnt_ref, cntc_ref, spos_ref, te_ref):
    cnt = cnt_ref[...]
    pad_cnt = jnp.floor((cnt + (T - 1)) * (1.0 / T)) * T
    e_r = lax.broadcasted_iota(jnp.int32, (E, E), 0)
    e_c = lax.broadcasted_iota(jnp.int32, (E, E), 1)
    strict_u = (e_r < e_c).astype(jnp.float32)
    base = lax.dot_general(pad_cnt, strict_u, (((1,), (0,)), ((), ())),
                           preferred_element_type=jnp.float32)
    ends = base + pad_cnt

    eio = lax.broadcasted_iota(jnp.int32, (NB, E), 1)
    oh0 = (topi_ref[:, 0:1] == eio).astype(jnp.float32)
    oh1 = (topi_ref[:, 1:2] == eio).astype(jnp.float32)
    s0 = rank_ref[:, 0:1] + jnp.sum(oh0 * base, axis=1, keepdims=True)
    s1 = rank_ref[:, 1:2] + jnp.sum(oh1 * base, axis=1, keepdims=True)
    spos = jnp.where(eio == 0, s0, jnp.where(eio == 1, s1, 0.0))
    spos_ref[...] = spos.astype(jnp.int32)

    tio = (lax.broadcasted_iota(jnp.int32, (NT, E), 0) * T).astype(jnp.float32)
    ge = (tio >= ends).astype(jnp.float32)
    te = jnp.minimum(jnp.sum(ge, axis=1), float(E - 1))
    te_row = te[None, :]

    k_r = lax.broadcasted_iota(jnp.int32, (NT, NT), 0)
    t_c = lax.broadcasted_iota(jnp.int32, (NT, NT), 1)
    shift = (k_r == t_c - 1).astype(jnp.float32)
    low_i = (k_r <= t_c).astype(jnp.float32)
    te_prev = lax.dot_general(te_row, shift, (((1,), (0,)), ((), ())),
                              preferred_element_type=jnp.float32)
    tlane = lax.broadcasted_iota(jnp.int32, (1, NT), 1)
    fr = jnp.where((te_row != te_prev) | (tlane == 0), 1.0, 0.0)
    rid = lax.dot_general(fr, low_i, (((1,), (0,)), ((), ())),
                          preferred_element_type=jnp.float32) - 1.0
    pr = rid - 2.0 * jnp.floor(rid * 0.5)

    cntc = cntc_ref[...]
    pad_cnt_c = jnp.floor((cntc + (T - 1)) * (1.0 / T)) * T
    e_col = lax.broadcasted_iota(jnp.int32, (E, 1), 0).astype(jnp.float32)
    used = jnp.sum(pad_cnt_c)
    present = (pad_cnt_c > 0.0) | ((e_col == E - 1) & (used < float(S)))
    cand = jnp.where(present & (e_col > te_row), e_col, float(E))
    nxe = jnp.min(cand, axis=0)[None, :]
    hn = jnp.where(nxe < float(E), 1.0, 0.0)
    nxe = jnp.minimum(nxe, float(E - 1))

    r8 = lax.broadcasted_iota(jnp.int32, (8, NT), 0)
    aux = jnp.where(r8 == 0, jnp.broadcast_to(te_row, (8, NT)),
          jnp.where(r8 == 1, jnp.broadcast_to(fr, (8, NT)),
          jnp.where(r8 == 2, jnp.broadcast_to(pr, (8, NT)),
          jnp.where(r8 == 3, jnp.broadcast_to(nxe, (8, NT)),
          jnp.where(r8 == 4, jnp.broadcast_to(hn, (8, NT)), 0.0)))))
    te_ref[...] = aux.astype(jnp.int32)


def _slots(rank, topi, cnt):
    return pl.pallas_call(
        _slots_body,
        grid=(N // NB,),
        in_specs=[
            pl.BlockSpec((NB, E), lambda i: (i, 0)),
            pl.BlockSpec((NB, E), lambda i: (i, 0)),
            pl.BlockSpec((1, E), lambda i: (0, 0)),
            pl.BlockSpec((E, 1), lambda i: (0, 0)),
        ],
        out_specs=[
            pl.BlockSpec((NB, E), lambda i: (i, 0)),
            pl.BlockSpec((8, NT), lambda i: (0, 0)),
        ],
        out_shape=[
            jax.ShapeDtypeStruct((N, E), jnp.int32),
            jax.ShapeDtypeStruct((8, NT), jnp.int32),
        ],
    )(rank, topi, cnt, cnt.reshape(E, 1))


def _dispatch_scatter_sc(x1, idx_flat):
    info = plsc.get_sparse_core_info()
    nw = info.num_cores * info.num_subcores
    bpw = N // nw
    mesh = plsc.VectorSubcoreMesh(core_axis_name="c", subcore_axis_name="s")

    @functools.partial(
        pl.kernel,
        out_type=jax.ShapeDtypeStruct((S, D), jnp.float32),
        mesh=mesh,
        scratch_types=[
            pltpu.VMEM((bpw, D), jnp.float32),
            pltpu.VMEM((bpw,), jnp.int32),
            pltpu.VMEM((bpw,), jnp.int32),
            pltpu.SemaphoreType.DMA,
            pltpu.SemaphoreType.DMA,
        ],
    )
    def k(x_hbm, idx_hbm, xs_hbm, rows_v, i0_v, i1_v, sem0, sem1):
        wid = lax.axis_index("s") * info.num_cores + lax.axis_index("c")
        base = wid * bpw
        pltpu.sync_copy(x_hbm.at[pl.ds(base, bpw)], rows_v)
        pltpu.sync_copy(idx_hbm.at[pl.ds(base, bpw)], i0_v)
        pltpu.sync_copy(idx_hbm.at[pl.ds(N + base, bpw)], i1_v)
        c0 = pltpu.async_copy(rows_v, xs_hbm.at[i0_v], sem0)
        c1 = pltpu.async_copy(rows_v, xs_hbm.at[i1_v], sem1)
        c0.wait()
        c1.wait()

    return k(x1, idx_flat)


def _combine_gather_sc(xout, idx_flat):
    info = plsc.get_sparse_core_info()
    nw = info.num_cores * info.num_subcores
    bpw = N // nw
    mesh = plsc.VectorSubcoreMesh(core_axis_name="c", subcore_axis_name="s")

    @functools.partial(
        pl.kernel,
        out_type=[
            jax.ShapeDtypeStruct((N, D), jnp.float32),
            jax.ShapeDtypeStruct((N, D), jnp.float32),
        ],
        mesh=mesh,
        scratch_types=[
            pltpu.VMEM((bpw, D), jnp.float32),
            pltpu.VMEM((bpw, D), jnp.float32),
            pltpu.VMEM((bpw,), jnp.int32),
            pltpu.VMEM((bpw,), jnp.int32),
            pltpu.SemaphoreType.DMA,
            pltpu.SemaphoreType.DMA,
        ],
    )
    def k(xo_hbm, idx_hbm, g0_hbm, g1_hbm, r0_v, r1_v, i0_v, i1_v, sem0, sem1):
        wid = lax.axis_index("s") * info.num_cores + lax.axis_index("c")
        base = wid * bpw
        pltpu.sync_copy(idx_hbm.at[pl.ds(base, bpw)], i0_v)
        pltpu.sync_copy(idx_hbm.at[pl.ds(N + base, bpw)], i1_v)
        c0 = pltpu.async_copy(xo_hbm.at[i0_v], r0_v, sem0)
        c1 = pltpu.async_copy(xo_hbm.at[i1_v], r1_v, sem1)
        c0.wait()
        c1.wait()
        pltpu.sync_copy(r0_v, g0_hbm.at[pl.ds(base, bpw)])
        pltpu.sync_copy(r1_v, g1_hbm.at[pl.ds(base, bpw)])

    return k(xout, idx_flat)


def _ffn_body(te_ref, fr_ref, pr_ref, nxe_ref, hn_ref,
              xs_ref, b1_ref, b2_ref, bo_ref, w1_hbm, w2_hbm, wo_hbm,
              out_ref, w1s_ref, w2s_ref, wos_ref,
              w1b_ref, w2b_ref, wob_ref, sem):
    j = pl.program_id(0)
    i = pl.program_id(1)

    def _issue(e_, slot, jj=None):
        jb = j if jj is None else jj
        pltpu.make_async_copy(
            w1_hbm.at[e_, pl.ds(jb * FB, FB), :], w1s_ref.at[slot],
            sem.at[0, slot]).start()
        pltpu.make_async_copy(
            w2_hbm.at[e_, pl.ds(jb * FB, FB), :], w2s_ref.at[slot],
            sem.at[1, slot]).start()
        pltpu.make_async_copy(
            wo_hbm.at[e_, :, pl.ds(jb * FB, FB)], wos_ref.at[slot],
            sem.at[2, slot]).start()

    def _wait(e_, slot):
        pltpu.make_async_copy(
            w1_hbm.at[e_, pl.ds(j * FB, FB), :], w1s_ref.at[slot],
            sem.at[0, slot]).wait()
        pltpu.make_async_copy(
            w2_hbm.at[e_, pl.ds(j * FB, FB), :], w2s_ref.at[slot],
            sem.at[1, slot]).wait()
        pltpu.make_async_copy(
            wo_hbm.at[e_, :, pl.ds(j * FB, FB)], wos_ref.at[slot],
            sem.at[2, slot]).wait()

    @pl.when((i == 0) & (j == 0))
    def _():
        _issue(te_ref[0], 0)

    @pl.when(fr_ref[i] == 1)
    def _():
        slot = pr_ref[i]

        @pl.when(slot == 0)
        def _():
            _wait(te_ref[i], 0)
            w1b_ref[...] = w1s_ref[0].astype(jnp.bfloat16)
            w2b_ref[...] = w2s_ref[0].astype(jnp.bfloat16)
            wob_ref[...] = wos_ref[0].astype(jnp.bfloat16)

            @pl.when(hn_ref[i] == 1)
            def _():
                _issue(nxe_ref[i], 1)

            @pl.when((hn_ref[i] == 0) & (j < NJ - 1))
            def _():
                _issue(te_ref[0], 0, jj=j + 1)

        @pl.when(slot == 1)
        def _():
            _wait(te_ref[i], 1)
            w1b_ref[...] = w1s_ref[1].astype(jnp.bfloat16)
            w2b_ref[...] = w2s_ref[1].astype(jnp.bfloat16)
            wob_ref[...] = wos_ref[1].astype(jnp.bfloat16)

            @pl.when(hn_ref[i] == 1)
            def _():
                _issue(nxe_ref[i], 0)

            @pl.when((hn_ref[i] == 0) & (j < NJ - 1))
            def _():
                _issue(te_ref[0], 0, jj=j + 1)

    x = xs_ref[...].astype(jnp.bfloat16)
    h1 = lax.dot_general(x, w1b_ref[...], (((1,), (1,)), ((), ())),
                         preferred_element_type=jnp.float32) + b1_ref[0]
    h2 = lax.dot_general(x, w2b_ref[...], (((1,), (1,)), ((), ())),
                         preferred_element_type=jnp.float32) + b2_ref[0]
    g = (h1 / (1.0 + jnp.exp(-h1))) * h2
    g = g.astype(jnp.bfloat16)
    acc = lax.dot_general(g, wob_ref[...], (((1,), (1,)), ((), ())),
                          preferred_element_type=jnp.float32)

    @pl.when(j == 0)
    def _():
        out_ref[pl.ds(i * T, T), :] = acc + bo_ref[0]

    @pl.when(j > 0)
    def _():
        out_ref[pl.ds(i * T, T), :] += acc


def _grouped_ffn(te, fr, pr, nxe, hn, xs, W1, b1, W2, b2, Wout, bout):
    grid_spec = pltpu.PrefetchScalarGridSpec(
        num_scalar_prefetch=5,
        grid=(NJ, NT),
        in_specs=[
            pl.BlockSpec((T, D), lambda j, i, *_: (i, 0)),
            pl.BlockSpec((1, 1, FB),
                         lambda j, i, te, *_: (te[i] * NJ + j, 0, 0)),
            pl.BlockSpec((1, 1, FB),
                         lambda j, i, te, *_: (te[i] * NJ + j, 0, 0)),
            pl.BlockSpec((1, 1, D), lambda j, i, te, *_: (te[i], 0, 0)),
            pl.BlockSpec(memory_space=pltpu.MemorySpace.HBM),
            pl.BlockSpec(memory_space=pltpu.MemorySpace.HBM),
            pl.BlockSpec(memory_space=pltpu.MemorySpace.HBM),
        ],
        out_specs=pl.BlockSpec((S, D), lambda j, i, *_: (0, 0)),
        scratch_shapes=[
            pltpu.VMEM((2, FB, D), jnp.float32),
            pltpu.VMEM((2, FB, D), jnp.float32),
            pltpu.VMEM((2, D, FB), jnp.float32),
            pltpu.VMEM((FB, D), jnp.bfloat16),
            pltpu.VMEM((FB, D), jnp.bfloat16),
            pltpu.VMEM((D, FB), jnp.bfloat16),
            pltpu.SemaphoreType.DMA((3, 2)),
        ],
    )
    return pl.pallas_call(
        _ffn_body,
        grid_spec=grid_spec,
        out_shape=jax.ShapeDtypeStruct((S, D), jnp.float32),
    )(te, fr, pr, nxe, hn, xs, b1.reshape(E * NJ, 1, FB),
      b2.reshape(E * NJ, 1, FB), bout.reshape(E, 1, D), W1, W2, Wout)


def _final_body(x1_ref, g0_ref, g1_ref, tw_ref, n2w_ref, n2b_ref, o_ref):
    w0 = tw_ref[:, 0:1]
    w1 = tw_ref[:, 1:2]
    y = x1_ref[...] + w0 * g0_ref[...] + w1 * g1_ref[...]
    mu = jnp.mean(y, axis=1, keepdims=True)
    yc = y - mu
    var = jnp.mean(yc * yc, axis=1, keepdims=True)
    o_ref[...] = yc * lax.rsqrt(var + EPS) * n2w_ref[...] + n2b_ref[...]


def _final(x1, g0, g1, topw, norm2_w, norm2_b):
    return pl.pallas_call(
        _final_body,
        grid=(N // NB,),
        in_specs=[
            pl.BlockSpec((NB, D), lambda i: (i, 0)),
            pl.BlockSpec((NB, D), lambda i: (i, 0)),
            pl.BlockSpec((NB, D), lambda i: (i, 0)),
            pl.BlockSpec((NB, E), lambda i: (i, 0)),
            pl.BlockSpec((1, D), lambda i: (0, 0)),
            pl.BlockSpec((1, D), lambda i: (0, 0)),
        ],
        out_specs=pl.BlockSpec((NB, D), lambda i: (i, 0)),
        out_shape=jax.ShapeDtypeStruct((N, D), jnp.float32),
    )(x1, g0, g1, topw, norm2_w.reshape(1, D), norm2_b.reshape(1, D))


def kernel(src, in_proj_w, in_proj_b, out_proj_w, out_proj_b,
           norm1_w, norm1_b, norm2_w, norm2_b,
           gate_w, gate_b, W1, b1, W2, b2, Wout, bout):
    qkvh = _qkv(src, in_proj_w, in_proj_b)
    oh = _attention(qkvh)
    x1, topi, topw = _post_attn(oh, src, out_proj_w, out_proj_b,
                                norm1_w, norm1_b, gate_w, gate_b)
    rank, cnt = _ranks(topi)
    spos, aux = _slots(rank, topi, cnt)
    idx_flat = jnp.concatenate([spos[:, 0], spos[:, 1]])
    xs = _dispatch_scatter_sc(x1, idx_flat)
    xout = _grouped_ffn(aux[0], aux[1], aux[2], aux[3], aux[4],
                        xs, W1, b1, W2, b2, Wout, bout)
    g0, g1 = _combine_gather_sc(xout, idx_flat)
    return _final(x1, g0, g1, topw, norm2_w, norm2_b)


# speedup vs baseline: 1.2071x; 1.0048x over previous
import functools

import jax
import jax.numpy as jnp
from jax import lax
from jax.experimental import pallas as pl
from jax.experimental.pallas import tpu as pltpu
from jax.experimental.pallas import tpu_sc as plsc

N = 2048
D = 768
H = 12
DH = D // H
FF = 3072
E = 8
K = 2
EPS = 1e-05

T = 256
S = N * K + E * T
NT = S // T
FB = 1024
NJ = FF // FB

NB = 256
QB = 2048


def _qkv_body(x_ref, w_ref, b_ref, o_ref, ws_ref):
    @pl.when(pl.program_id(0) == 0)
    def _():
        ws_ref[...] = w_ref[...].astype(jnp.bfloat16)

    x = x_ref[...].astype(jnp.bfloat16)
    acc = lax.dot_general(ws_ref[...], x, (((1,), (1,)), ((), ())),
                          preferred_element_type=jnp.float32)
    acc = acc + b_ref[...]
    rio = lax.broadcasted_iota(jnp.int32, (3 * D, 1), 0)
    acc = acc * jnp.where(rio < D, 1.0 / (DH ** 0.5), 1.0)
    o_ref[...] = acc.astype(jnp.bfloat16).reshape(3 * H, DH, NB)


def _qkv(src, in_proj_w, in_proj_b):
    return pl.pallas_call(
        _qkv_body,
        grid=(N // NB,),
        in_specs=[
            pl.BlockSpec((NB, D), lambda i: (i, 0)),
            pl.BlockSpec((3 * D, D), lambda i: (0, 0)),
            pl.BlockSpec((3 * D, 1), lambda i: (0, 0)),
        ],
        out_specs=pl.BlockSpec((3 * H, DH, NB), lambda i: (0, 0, i)),
        out_shape=jax.ShapeDtypeStruct((3 * H, DH, N), jnp.bfloat16),
        scratch_shapes=[pltpu.VMEM((3 * D, D), jnp.bfloat16)],
    )(src, in_proj_w, in_proj_b.reshape(3 * D, 1))


def _attn_body(q_ref, k_ref, v_ref, o_ref):
    qt = q_ref[0]
    kt = k_ref[0]
    s = lax.dot_general(qt, kt, (((0,), (0,)), ((), ())),
                        preferred_element_type=jnp.float32)
    m = jnp.max(s, axis=1, keepdims=True)
    p = jnp.exp(s - m)
    l = jnp.sum(p, axis=1, keepdims=True)
    o = lax.dot_general(p.astype(jnp.bfloat16), v_ref[0],
                        (((1,), (1,)), ((), ())),
                        preferred_element_type=jnp.float32)
    o_ref[0] = (o * (1.0 / l)).astype(jnp.bfloat16)


def _attention(qkvt):
    return pl.pallas_call(
        _attn_body,
        grid=(H, N // QB),
        in_specs=[
            pl.BlockSpec((1, DH, QB), lambda h, i: (h, 0, i)),
            pl.BlockSpec((1, DH, N), lambda h, i: (H + h, 0, 0)),
            pl.BlockSpec((1, DH, N), lambda h, i: (2 * H + h, 0, 0)),
        ],
        out_specs=pl.BlockSpec((1, QB, DH), lambda h, i: (h, i, 0)),
        out_shape=jax.ShapeDtypeStruct((H, N, DH), jnp.bfloat16),
    )(qkvt, qkvt, qkvt)


def _post_attn_body(o_ref, src_ref, wo_ref, bo_ref, n1w_ref, n1b_ref,
                    gw_ref, gb_ref, x1_ref, topi_ref, topw_ref, ws_ref):
    @pl.when(pl.program_id(0) == 0)
    def _():
        ws_ref[...] = wo_ref[...].astype(jnp.bfloat16)

    o = jnp.transpose(o_ref[...], (1, 0, 2)).reshape(NB, D)
    sa = lax.dot_general(o, ws_ref[...], (((1,), (1,)), ((), ())),
                         preferred_element_type=jnp.float32)
    y = sa + bo_ref[...] + src_ref[...]
    mu = jnp.mean(y, axis=1, keepdims=True)
    yc = y - mu
    var = jnp.mean(yc * yc, axis=1, keepdims=True)
    x1 = yc * lax.rsqrt(var + EPS) * n1w_ref[...] + n1b_ref[...]
    x1_ref[...] = x1

    logits = lax.dot_general(x1, gw_ref[...], (((1,), (1,)), ((), ())),
                             preferred_element_type=jnp.float32) + gb_ref[...]
    lm = jnp.max(logits, axis=1, keepdims=True)
    eg = jnp.exp(logits - lm)
    g = eg / jnp.sum(eg, axis=1, keepdims=True)
    eio = lax.broadcasted_iota(jnp.int32, (NB, E), 1)
    m1 = jnp.max(g, axis=1, keepdims=True)
    i1 = jnp.min(jnp.where(g == m1, eio, E), axis=1, keepdims=True)
    g2 = jnp.where(eio == i1, -1.0, g)
    m2 = jnp.max(g2, axis=1, keepdims=True)
    i2 = jnp.min(jnp.where(g2 == m2, eio, E), axis=1, keepdims=True)
    topi_ref[...] = jnp.where(eio == 0, i1, jnp.where(eio == 1, i2, 0))
    topw_ref[...] = jnp.where(eio == 0, m1, jnp.where(eio == 1, m2, 0.0))


def _post_attn(o, src, out_proj_w, out_proj_b, norm1_w, norm1_b, gate_w, gate_b):
    return pl.pallas_call(
        _post_attn_body,
        grid=(N // NB,),
        in_specs=[
            pl.BlockSpec((H, NB, DH), lambda i: (0, i, 0)),
            pl.BlockSpec((NB, D), lambda i: (i, 0)),
            pl.BlockSpec((D, D), lambda i: (0, 0)),
            pl.BlockSpec((1, D), lambda i: (0, 0)),
            pl.BlockSpec((1, D), lambda i: (0, 0)),
            pl.BlockSpec((1, D), lambda i: (0, 0)),
            pl.BlockSpec((E, D), lambda i: (0, 0)),
            pl.BlockSpec((1, E), lambda i: (0, 0)),
        ],
        out_specs=[
            pl.BlockSpec((NB, D), lambda i: (i, 0)),
            pl.BlockSpec((NB, E), lambda i: (i, 0)),
            pl.BlockSpec((NB, E), lambda i: (i, 0)),
        ],
        out_shape=[
            jax.ShapeDtypeStruct((N, D), jnp.float32),
            jax.ShapeDtypeStruct((N, E), jnp.int32),
            jax.ShapeDtypeStruct((N, E), jnp.float32),
        ],
        scratch_shapes=[pltpu.VMEM((D, D), jnp.bfloat16)],
    )(o, src, out_proj_w, out_proj_b.reshape(1, D), norm1_w.reshape(1, D),
      norm1_b.reshape(1, D), gate_w, gate_b.reshape(1, E))


def _rank_body(topi_ref, rank_ref, cnt_ref, tot_ref):
    i = pl.program_id(0)

    @pl.when(i == 0)
    def _():
        tot_ref[...] = jnp.zeros_like(tot_ref)

    eio = lax.broadcasted_iota(jnp.int32, (NB, E), 1)
    oh0 = (topi_ref[:, 0:1] == eio).astype(jnp.float32)
    oh1 = (topi_ref[:, 1:2] == eio).astype(jnp.float32)
    c = oh0 + oh1
    r = lax.broadcasted_iota(jnp.int32, (NB, NB), 0)
    cc = lax.broadcasted_iota(jnp.int32, (NB, NB), 1)
    strict_l = (r > cc).astype(jnp.float32)
    pre = lax.dot_general(strict_l, c, (((1,), (0,)), ((), ())),
                          preferred_element_type=jnp.float32)
    base = tot_ref[...] + pre
    r0 = jnp.sum(oh0 * base, axis=1, keepdims=True)
    r1 = jnp.sum(oh1 * base, axis=1, keepdims=True)
    eiof = lax.broadcasted_iota(jnp.int32, (NB, E), 1)
    rank_ref[...] = jnp.where(eiof == 0, r0, jnp.where(eiof == 1, r1, 0.0))
    tot_ref[...] = tot_ref[0:1, :] + jnp.sum(c, axis=0, keepdims=True)
    cnt_ref[...] = tot_ref[0:1, :]


def _ranks(topi):
    return pl.pallas_call(
        _rank_body,
        grid=(N // NB,),
        in_specs=[pl.BlockSpec((NB, E), lambda i: (i, 0))],
        out_specs=[
            pl.BlockSpec((NB, E), lambda i: (i, 0)),
            pl.BlockSpec((1, E), lambda i: (0, 0)),
        ],
        out_shape=[
            jax.ShapeDtypeStruct((N, E), jnp.float32),
            jax.ShapeDtypeStruct((1, E), jnp.float32),
        ],
        scratch_shapes=[pltpu.VMEM((1, E), jnp.float32)],
    )(topi)


def _slots_body(rank_ref, topi_ref, cnt_ref, cntc_ref, spos_ref, te_ref):
    cnt = cnt_ref[...]
    pad_cnt = jnp.floor((cnt + (T - 1)) * (1.0 / T)) * T
    e_r = lax.broadcasted_iota(jnp.int32, (E, E), 0)
    e_c = lax.broadcasted_iota(jnp.int32, (E, E), 1)
    strict_u = (e_r < e_c).astype(jnp.float32)
    base = lax.dot_general(pad_cnt, strict_u, (((1,), (0,)), ((), ())),
                           preferred_element_type=jnp.float32)
    ends = base + pad_cnt

    eio = lax.broadcasted_iota(jnp.int32, (NB, E), 1)
    oh0 = (topi_ref[:, 0:1] == eio).astype(jnp.float32)
    oh1 = (topi_ref[:, 1:2] == eio).astype(jnp.float32)
    s0 = rank_ref[:, 0:1] + jnp.sum(oh0 * base, axis=1, keepdims=True)
    s1 = rank_ref[:, 1:2] + jnp.sum(oh1 * base, axis=1, keepdims=True)
    rio2 = lax.broadcasted_iota(jnp.int32, (2, NB), 0)
    spos_ref[...] = jnp.where(
        rio2 == 0, s0.reshape(1, NB), s1.reshape(1, NB)).astype(jnp.int32)

    tio = (lax.broadcasted_iota(jnp.int32, (NT, E), 0) * T).astype(jnp.float32)
    ge = (tio >= ends).astype(jnp.float32)
    te = jnp.minimum(jnp.sum(ge, axis=1), float(E - 1))
    te_row = te[None, :]

    k_r = lax.broadcasted_iota(jnp.int32, (NT, NT), 0)
    t_c = lax.broadcasted_iota(jnp.int32, (NT, NT), 1)
    shift = (k_r == t_c - 1).astype(jnp.float32)
    low_i = (k_r <= t_c).astype(jnp.float32)
    te_prev = lax.dot_general(te_row, shift, (((1,), (0,)), ((), ())),
                              preferred_element_type=jnp.float32)
    tlane = lax.broadcasted_iota(jnp.int32, (1, NT), 1)
    fr = jnp.where((te_row != te_prev) | (tlane == 0), 1.0, 0.0)
    rid = lax.dot_general(fr, low_i, (((1,), (0,)), ((), ())),
                          preferred_element_type=jnp.float32) - 1.0
    pr = rid - 2.0 * jnp.floor(rid * 0.5)

    cntc = cntc_ref[...]
    pad_cnt_c = jnp.floor((cntc + (T - 1)) * (1.0 / T)) * T
    e_col = lax.broadcasted_iota(jnp.int32, (E, 1), 0).astype(jnp.float32)
    used = jnp.sum(pad_cnt_c)
    present = (pad_cnt_c > 0.0) | ((e_col == E - 1) & (used < float(S)))
    cand = jnp.where(present & (e_col > te_row), e_col, float(E))
    nxe = jnp.min(cand, axis=0)[None, :]
    hn = jnp.where(nxe < float(E), 1.0, 0.0)
    nxe = jnp.minimum(nxe, float(E - 1))

    r8 = lax.broadcasted_iota(jnp.int32, (8, NT), 0)
    aux = jnp.where(r8 == 0, jnp.broadcast_to(te_row, (8, NT)),
          jnp.where(r8 == 1, jnp.broadcast_to(fr, (8, NT)),
          jnp.where(r8 == 2, jnp.broadcast_to(pr, (8, NT)),
          jnp.where(r8 == 3, jnp.broadcast_to(nxe, (8, NT)),
          jnp.where(r8 == 4, jnp.broadcast_to(hn, (8, NT)), 0.0)))))
    te_ref[...] = aux.astype(jnp.int32)


def _slots(rank, topi, cnt):
    return pl.pallas_call(
        _slots_body,
        grid=(N // NB,),
        in_specs=[
            pl.BlockSpec((NB, E), lambda i: (i, 0)),
            pl.BlockSpec((NB, E), lambda i: (i, 0)),
            pl.BlockSpec((1, E), lambda i: (0, 0)),
            pl.BlockSpec((E, 1), lambda i: (0, 0)),
        ],
        out_specs=[
            pl.BlockSpec((2, NB), lambda i: (0, i)),
            pl.BlockSpec((8, NT), lambda i: (0, 0)),
        ],
        out_shape=[
            jax.ShapeDtypeStruct((2, N), jnp.int32),
            jax.ShapeDtypeStruct((8, NT), jnp.int32),
        ],
    )(rank, topi, cnt, cnt.reshape(E, 1))


def _dispatch_scatter_sc(x1, idx_flat):
    info = plsc.get_sparse_core_info()
    nw = info.num_cores * info.num_subcores
    bpw = N // nw
    mesh = plsc.VectorSubcoreMesh(core_axis_name="c", subcore_axis_name="s")

    @functools.partial(
        pl.kernel,
        out_type=jax.ShapeDtypeStruct((S, D), jnp.float32),
        mesh=mesh,
        scratch_types=[
            pltpu.VMEM((bpw, D), jnp.float32),
            pltpu.VMEM((bpw,), jnp.int32),
            pltpu.VMEM((bpw,), jnp.int32),
            pltpu.SemaphoreType.DMA,
            pltpu.SemaphoreType.DMA,
        ],
    )
    def k(x_hbm, idx_hbm, xs_hbm, rows_v, i0_v, i1_v, sem0, sem1):
        wid = lax.axis_index("s") * info.num_cores + lax.axis_index("c")
        base = wid * bpw
        pltpu.sync_copy(x_hbm.at[pl.ds(base, bpw)], rows_v)
        pltpu.sync_copy(idx_hbm.at[pl.ds(base, bpw)], i0_v)
        pltpu.sync_copy(idx_hbm.at[pl.ds(N + base, bpw)], i1_v)
        c0 = pltpu.async_copy(rows_v, xs_hbm.at[i0_v], sem0)
        c1 = pltpu.async_copy(rows_v, xs_hbm.at[i1_v], sem1)
        c0.wait()
        c1.wait()

    return k(x1, idx_flat)


def _combine_gather_sc(xout, idx_flat):
    info = plsc.get_sparse_core_info()
    nw = info.num_cores * info.num_subcores
    bpw = N // nw
    mesh = plsc.VectorSubcoreMesh(core_axis_name="c", subcore_axis_name="s")

    @functools.partial(
        pl.kernel,
        out_type=[
            jax.ShapeDtypeStruct((N, D), jnp.float32),
            jax.ShapeDtypeStruct((N, D), jnp.float32),
        ],
        mesh=mesh,
        scratch_types=[
            pltpu.VMEM((bpw, D), jnp.float32),
            pltpu.VMEM((bpw, D), jnp.float32),
            pltpu.VMEM((bpw,), jnp.int32),
            pltpu.VMEM((bpw,), jnp.int32),
            pltpu.SemaphoreType.DMA,
            pltpu.SemaphoreType.DMA,
        ],
    )
    def k(xo_hbm, idx_hbm, g0_hbm, g1_hbm, r0_v, r1_v, i0_v, i1_v, sem0, sem1):
        wid = lax.axis_index("s") * info.num_cores + lax.axis_index("c")
        base = wid * bpw
        pltpu.sync_copy(idx_hbm.at[pl.ds(base, bpw)], i0_v)
        pltpu.sync_copy(idx_hbm.at[pl.ds(N + base, bpw)], i1_v)
        c0 = pltpu.async_copy(xo_hbm.at[i0_v], r0_v, sem0)
        c1 = pltpu.async_copy(xo_hbm.at[i1_v], r1_v, sem1)
        c0.wait()
        c1.wait()
        pltpu.sync_copy(r0_v, g0_hbm.at[pl.ds(base, bpw)])
        pltpu.sync_copy(r1_v, g1_hbm.at[pl.ds(base, bpw)])

    return k(xout, idx_flat)


def _ffn_body(te_ref, fr_ref, pr_ref, nxe_ref, hn_ref,
              xs_ref, b1_ref, b2_ref, bo_ref, w1_hbm, w2_hbm, wo_hbm,
              out_ref, w1s_ref, w2s_ref, wos_ref,
              w1b_ref, w2b_ref, wob_ref, sem):
    j = pl.program_id(0)
    i = pl.program_id(1)

    def _issue(e_, slot, jj=None):
        jb = j if jj is None else jj
        pltpu.make_async_copy(
            w1_hbm.at[e_, pl.ds(jb * FB, FB), :], w1s_ref.at[slot],
            sem.at[0, slot]).start()
        pltpu.make_async_copy(
            w2_hbm.at[e_, pl.ds(jb * FB, FB), :], w2s_ref.at[slot],
            sem.at[1, slot]).start()
        pltpu.make_async_copy(
            wo_hbm.at[e_, :, pl.ds(jb * FB, FB)], wos_ref.at[slot],
            sem.at[2, slot]).start()

    def _wait(e_, slot):
        pltpu.make_async_copy(
            w1_hbm.at[e_, pl.ds(j * FB, FB), :], w1s_ref.at[slot],
            sem.at[0, slot]).wait()
        pltpu.make_async_copy(
            w2_hbm.at[e_, pl.ds(j * FB, FB), :], w2s_ref.at[slot],
            sem.at[1, slot]).wait()
        pltpu.make_async_copy(
            wo_hbm.at[e_, :, pl.ds(j * FB, FB)], wos_ref.at[slot],
            sem.at[2, slot]).wait()

    @pl.when((i == 0) & (j == 0))
    def _():
        _issue(te_ref[0], 0)

    @pl.when(fr_ref[i] == 1)
    def _():
        slot = pr_ref[i]

        @pl.when(slot == 0)
        def _():
            _wait(te_ref[i], 0)
            w1b_ref[...] = w1s_ref[0].astype(jnp.bfloat16)
            w2b_ref[...] = w2s_ref[0].astype(jnp.bfloat16)
            wob_ref[...] = wos_ref[0].astype(jnp.bfloat16)

            @pl.when(hn_ref[i] == 1)
            def _():
                _issue(nxe_ref[i], 1)

            @pl.when((hn_ref[i] == 0) & (j < NJ - 1))
            def _():
                _issue(te_ref[0], 0, jj=j + 1)

        @pl.when(slot == 1)
        def _():
            _wait(te_ref[i], 1)
            w1b_ref[...] = w1s_ref[1].astype(jnp.bfloat16)
            w2b_ref[...] = w2s_ref[1].astype(jnp.bfloat16)
            wob_ref[...] = wos_ref[1].astype(jnp.bfloat16)

            @pl.when(hn_ref[i] == 1)
            def _():
                _issue(nxe_ref[i], 0)

            @pl.when((hn_ref[i] == 0) & (j < NJ - 1))
            def _():
                _issue(te_ref[0], 0, jj=j + 1)

    x = xs_ref[...].astype(jnp.bfloat16)
    h1 = lax.dot_general(x, w1b_ref[...], (((1,), (1,)), ((), ())),
                         preferred_element_type=jnp.float32) + b1_ref[0]
    h2 = lax.dot_general(x, w2b_ref[...], (((1,), (1,)), ((), ())),
                         preferred_element_type=jnp.float32) + b2_ref[0]
    g = (h1 / (1.0 + jnp.exp(-h1))) * h2
    g = g.astype(jnp.bfloat16)
    acc = lax.dot_general(g, wob_ref[...], (((1,), (1,)), ((), ())),
                          preferred_element_type=jnp.float32)

    @pl.when(j == 0)
    def _():
        out_ref[pl.ds(i * T, T), :] = acc + bo_ref[0]

    @pl.when(j > 0)
    def _():
        out_ref[pl.ds(i * T, T), :] += acc


def _grouped_ffn(te, fr, pr, nxe, hn, xs, W1, b1, W2, b2, Wout, bout):
    grid_spec = pltpu.PrefetchScalarGridSpec(
        num_scalar_prefetch=5,
        grid=(NJ, NT),
        in_specs=[
            pl.BlockSpec((T, D), lambda j, i, *_: (i, 0)),
            pl.BlockSpec((1, 1, FB),
                         lambda j, i, te, *_: (te[i] * NJ + j, 0, 0)),
            pl.BlockSpec((1, 1, FB),
                         lambda j, i, te, *_: (te[i] * NJ + j, 0, 0)),
            pl.BlockSpec((1, 1, D), lambda j, i, te, *_: (te[i], 0, 0)),
            pl.BlockSpec(memory_space=pltpu.MemorySpace.HBM),
            pl.BlockSpec(memory_space=pltpu.MemorySpace.HBM),
            pl.BlockSpec(memory_space=pltpu.MemorySpace.HBM),
        ],
        out_specs=pl.BlockSpec((S, D), lambda j, i, *_: (0, 0)),
        scratch_shapes=[
            pltpu.VMEM((2, FB, D), jnp.float32),
            pltpu.VMEM((2, FB, D), jnp.float32),
            pltpu.VMEM((2, D, FB), jnp.float32),
            pltpu.VMEM((FB, D), jnp.bfloat16),
            pltpu.VMEM((FB, D), jnp.bfloat16),
            pltpu.VMEM((D, FB), jnp.bfloat16),
            pltpu.SemaphoreType.DMA((3, 2)),
        ],
    )
    return pl.pallas_call(
        _ffn_body,
        grid_spec=grid_spec,
        out_shape=jax.ShapeDtypeStruct((S, D), jnp.float32),
    )(te, fr, pr, nxe, hn, xs, b1.reshape(E * NJ, 1, FB),
      b2.reshape(E * NJ, 1, FB), bout.reshape(E, 1, D), W1, W2, Wout)


def _final_body(x1_ref, g0_ref, g1_ref, tw_ref, n2w_ref, n2b_ref, o_ref):
    w0 = tw_ref[:, 0:1]
    w1 = tw_ref[:, 1:2]
    y = x1_ref[...] + w0 * g0_ref[...] + w1 * g1_ref[...]
    mu = jnp.mean(y, axis=1, keepdims=True)
    yc = y - mu
    var = jnp.mean(yc * yc, axis=1, keepdims=True)
    o_ref[...] = yc * lax.rsqrt(var + EPS) * n2w_ref[...] + n2b_ref[...]


def _final(x1, g0, g1, topw, norm2_w, norm2_b):
    return pl.pallas_call(
        _final_body,
        grid=(N // NB,),
        in_specs=[
            pl.BlockSpec((NB, D), lambda i: (i, 0)),
            pl.BlockSpec((NB, D), lambda i: (i, 0)),
            pl.BlockSpec((NB, D), lambda i: (i, 0)),
            pl.BlockSpec((NB, E), lambda i: (i, 0)),
            pl.BlockSpec((1, D), lambda i: (0, 0)),
            pl.BlockSpec((1, D), lambda i: (0, 0)),
        ],
        out_specs=pl.BlockSpec((NB, D), lambda i: (i, 0)),
        out_shape=jax.ShapeDtypeStruct((N, D), jnp.float32),
    )(x1, g0, g1, topw, norm2_w.reshape(1, D), norm2_b.reshape(1, D))


def kernel(src, in_proj_w, in_proj_b, out_proj_w, out_proj_b,
           norm1_w, norm1_b, norm2_w, norm2_b,
           gate_w, gate_b, W1, b1, W2, b2, Wout, bout):
    qkvh = _qkv(src, in_proj_w, in_proj_b)
    oh = _attention(qkvh)
    x1, topi, topw = _post_attn(oh, src, out_proj_w, out_proj_b,
                                norm1_w, norm1_b, gate_w, gate_b)
    rank, cnt = _ranks(topi)
    spos, aux = _slots(rank, topi, cnt)
    idx_flat = spos.reshape(2 * N)
    xs = _dispatch_scatter_sc(x1, idx_flat)
    xout = _grouped_ffn(aux[0], aux[1], aux[2], aux[3], aux[4],
                        xs, W1, b1, W2, b2, Wout, bout)
    g0, g1 = _combine_gather_sc(xout, idx_flat)
    return _final(x1, g0, g1, topw, norm2_w, norm2_b)


# speedup vs baseline: 1.2243x; 1.0142x over previous
import functools

import jax
import jax.numpy as jnp
from jax import lax
from jax.experimental import pallas as pl
from jax.experimental.pallas import tpu as pltpu
from jax.experimental.pallas import tpu_sc as plsc

N = 2048
D = 768
H = 12
DH = D // H
FF = 3072
E = 8
K = 2
EPS = 1e-05

T = 256
S = N * K + E * T
NT = S // T
FB = 1024
NJ = FF // FB

NB = 256
QB = 2048


def _qkv_body(x_ref, w_ref, b_ref, o_ref, ws_ref):
    @pl.when(pl.program_id(0) == 0)
    def _():
        ws_ref[...] = w_ref[...].astype(jnp.bfloat16)

    x = x_ref[...].astype(jnp.bfloat16)
    acc = lax.dot_general(ws_ref[...], x, (((1,), (1,)), ((), ())),
                          preferred_element_type=jnp.float32)
    acc = acc + b_ref[...]
    rio = lax.broadcasted_iota(jnp.int32, (3 * D, 1), 0)
    acc = acc * jnp.where(rio < D, 1.0 / (DH ** 0.5), 1.0)
    o_ref[...] = acc.astype(jnp.bfloat16).reshape(3 * H, DH, NB)


def _qkv(src, in_proj_w, in_proj_b):
    return pl.pallas_call(
        _qkv_body,
        grid=(N // NB,),
        in_specs=[
            pl.BlockSpec((NB, D), lambda i: (i, 0)),
            pl.BlockSpec((3 * D, D), lambda i: (0, 0)),
            pl.BlockSpec((3 * D, 1), lambda i: (0, 0)),
        ],
        out_specs=pl.BlockSpec((3 * H, DH, NB), lambda i: (0, 0, i)),
        out_shape=jax.ShapeDtypeStruct((3 * H, DH, N), jnp.bfloat16),
        scratch_shapes=[pltpu.VMEM((3 * D, D), jnp.bfloat16)],
    )(src, in_proj_w, in_proj_b.reshape(3 * D, 1))


def _attn_body(q_ref, k_ref, v_ref, o_ref):
    qt = q_ref[0]
    kt = k_ref[0]
    s = lax.dot_general(qt, kt, (((0,), (0,)), ((), ())),
                        preferred_element_type=jnp.float32)
    m = jnp.max(s, axis=1, keepdims=True)
    p = jnp.exp(s - m)
    l = jnp.sum(p, axis=1, keepdims=True)
    o = lax.dot_general(p.astype(jnp.bfloat16), v_ref[0],
                        (((1,), (1,)), ((), ())),
                        preferred_element_type=jnp.float32)
    o_ref[0] = (o * (1.0 / l)).astype(jnp.bfloat16)


def _attention(qkvt):
    return pl.pallas_call(
        _attn_body,
        grid=(H, N // QB),
        in_specs=[
            pl.BlockSpec((1, DH, QB), lambda h, i: (h, 0, i)),
            pl.BlockSpec((1, DH, N), lambda h, i: (H + h, 0, 0)),
            pl.BlockSpec((1, DH, N), lambda h, i: (2 * H + h, 0, 0)),
        ],
        out_specs=pl.BlockSpec((1, QB, DH), lambda h, i: (h, i, 0)),
        out_shape=jax.ShapeDtypeStruct((H, N, DH), jnp.bfloat16),
    )(qkvt, qkvt, qkvt)


def _post_attn_body(o_ref, src_ref, wo_ref, bo_ref, n1w_ref, n1b_ref,
                    gw_ref, gb_ref, x1_ref, topi_ref, topw_ref,
                    rank_ref, cnt_ref, ws_ref, tot_ref):
    @pl.when(pl.program_id(0) == 0)
    def _():
        ws_ref[...] = wo_ref[...].astype(jnp.bfloat16)
        tot_ref[...] = jnp.zeros_like(tot_ref)

    o = jnp.transpose(o_ref[...], (1, 0, 2)).reshape(NB, D)
    sa = lax.dot_general(o, ws_ref[...], (((1,), (1,)), ((), ())),
                         preferred_element_type=jnp.float32)
    y = sa + bo_ref[...] + src_ref[...]
    mu = jnp.mean(y, axis=1, keepdims=True)
    yc = y - mu
    var = jnp.mean(yc * yc, axis=1, keepdims=True)
    x1 = yc * lax.rsqrt(var + EPS) * n1w_ref[...] + n1b_ref[...]
    x1_ref[...] = x1

    logits = lax.dot_general(x1, gw_ref[...], (((1,), (1,)), ((), ())),
                             preferred_element_type=jnp.float32) + gb_ref[...]
    lm = jnp.max(logits, axis=1, keepdims=True)
    eg = jnp.exp(logits - lm)
    g = eg / jnp.sum(eg, axis=1, keepdims=True)
    eio = lax.broadcasted_iota(jnp.int32, (NB, E), 1)
    m1 = jnp.max(g, axis=1, keepdims=True)
    i1 = jnp.min(jnp.where(g == m1, eio, E), axis=1, keepdims=True)
    g2 = jnp.where(eio == i1, -1.0, g)
    m2 = jnp.max(g2, axis=1, keepdims=True)
    i2 = jnp.min(jnp.where(g2 == m2, eio, E), axis=1, keepdims=True)
    topi_ref[...] = jnp.where(eio == 0, i1, jnp.where(eio == 1, i2, 0))
    topw_ref[...] = jnp.where(eio == 0, m1, jnp.where(eio == 1, m2, 0.0))

    oh0 = (i1 == eio).astype(jnp.float32)
    oh1 = (i2 == eio).astype(jnp.float32)
    c = oh0 + oh1
    r = lax.broadcasted_iota(jnp.int32, (NB, NB), 0)
    cc = lax.broadcasted_iota(jnp.int32, (NB, NB), 1)
    strict_l = (r > cc).astype(jnp.float32)
    pre = lax.dot_general(strict_l, c, (((1,), (0,)), ((), ())),
                          preferred_element_type=jnp.float32)
    cbase = tot_ref[...] + pre
    r0 = jnp.sum(oh0 * cbase, axis=1, keepdims=True)
    r1 = jnp.sum(oh1 * cbase, axis=1, keepdims=True)
    rank_ref[...] = jnp.where(eio == 0, r0, jnp.where(eio == 1, r1, 0.0))
    tot_ref[...] = tot_ref[0:1, :] + jnp.sum(c, axis=0, keepdims=True)
    cnt_ref[...] = tot_ref[0:1, :]


def _post_attn(o, src, out_proj_w, out_proj_b, norm1_w, norm1_b, gate_w, gate_b):
    return pl.pallas_call(
        _post_attn_body,
        grid=(N // NB,),
        in_specs=[
            pl.BlockSpec((H, NB, DH), lambda i: (0, i, 0)),
            pl.BlockSpec((NB, D), lambda i: (i, 0)),
            pl.BlockSpec((D, D), lambda i: (0, 0)),
            pl.BlockSpec((1, D), lambda i: (0, 0)),
            pl.BlockSpec((1, D), lambda i: (0, 0)),
            pl.BlockSpec((1, D), lambda i: (0, 0)),
            pl.BlockSpec((E, D), lambda i: (0, 0)),
            pl.BlockSpec((1, E), lambda i: (0, 0)),
        ],
        out_specs=[
            pl.BlockSpec((NB, D), lambda i: (i, 0)),
            pl.BlockSpec((NB, E), lambda i: (i, 0)),
            pl.BlockSpec((NB, E), lambda i: (i, 0)),
            pl.BlockSpec((NB, E), lambda i: (i, 0)),
            pl.BlockSpec((1, E), lambda i: (0, 0)),
        ],
        out_shape=[
            jax.ShapeDtypeStruct((N, D), jnp.float32),
            jax.ShapeDtypeStruct((N, E), jnp.int32),
            jax.ShapeDtypeStruct((N, E), jnp.float32),
            jax.ShapeDtypeStruct((N, E), jnp.float32),
            jax.ShapeDtypeStruct((1, E), jnp.float32),
        ],
        scratch_shapes=[pltpu.VMEM((D, D), jnp.bfloat16),
                        pltpu.VMEM((1, E), jnp.float32)],
    )(o, src, out_proj_w, out_proj_b.reshape(1, D), norm1_w.reshape(1, D),
      norm1_b.reshape(1, D), gate_w, gate_b.reshape(1, E))


def _slots_body(rank_ref, topi_ref, cnt_ref, cntc_ref, spos_ref, te_ref):
    cnt = cnt_ref[...]
    pad_cnt = jnp.floor((cnt + (T - 1)) * (1.0 / T)) * T
    e_r = lax.broadcasted_iota(jnp.int32, (E, E), 0)
    e_c = lax.broadcasted_iota(jnp.int32, (E, E), 1)
    strict_u = (e_r < e_c).astype(jnp.float32)
    base = lax.dot_general(pad_cnt, strict_u, (((1,), (0,)), ((), ())),
                           preferred_element_type=jnp.float32)
    ends = base + pad_cnt

    eio = lax.broadcasted_iota(jnp.int32, (NB, E), 1)
    oh0 = (topi_ref[:, 0:1] == eio).astype(jnp.float32)
    oh1 = (topi_ref[:, 1:2] == eio).astype(jnp.float32)
    s0 = rank_ref[:, 0:1] + jnp.sum(oh0 * base, axis=1, keepdims=True)
    s1 = rank_ref[:, 1:2] + jnp.sum(oh1 * base, axis=1, keepdims=True)
    rio2 = lax.broadcasted_iota(jnp.int32, (2, NB), 0)
    spos_ref[...] = jnp.where(
        rio2 == 0, s0.reshape(1, NB), s1.reshape(1, NB)).astype(jnp.int32)

    tio = (lax.broadcasted_iota(jnp.int32, (NT, E), 0) * T).astype(jnp.float32)
    ge = (tio >= ends).astype(jnp.float32)
    te = jnp.minimum(jnp.sum(ge, axis=1), float(E - 1))
    te_row = te[None, :]

    k_r = lax.broadcasted_iota(jnp.int32, (NT, NT), 0)
    t_c = lax.broadcasted_iota(jnp.int32, (NT, NT), 1)
    shift = (k_r == t_c - 1).astype(jnp.float32)
    low_i = (k_r <= t_c).astype(jnp.float32)
    te_prev = lax.dot_general(te_row, shift, (((1,), (0,)), ((), ())),
                              preferred_element_type=jnp.float32)
    tlane = lax.broadcasted_iota(jnp.int32, (1, NT), 1)
    fr = jnp.where((te_row != te_prev) | (tlane == 0), 1.0, 0.0)
    rid = lax.dot_general(fr, low_i, (((1,), (0,)), ((), ())),
                          preferred_element_type=jnp.float32) - 1.0
    pr = rid - 2.0 * jnp.floor(rid * 0.5)

    cntc = cntc_ref[...]
    pad_cnt_c = jnp.floor((cntc + (T - 1)) * (1.0 / T)) * T
    e_col = lax.broadcasted_iota(jnp.int32, (E, 1), 0).astype(jnp.float32)
    used = jnp.sum(pad_cnt_c)
    present = (pad_cnt_c > 0.0) | ((e_col == E - 1) & (used < float(S)))
    cand = jnp.where(present & (e_col > te_row), e_col, float(E))
    nxe = jnp.min(cand, axis=0)[None, :]
    hn = jnp.where(nxe < float(E), 1.0, 0.0)
    nxe = jnp.minimum(nxe, float(E - 1))

    r8 = lax.broadcasted_iota(jnp.int32, (8, NT), 0)
    aux = jnp.where(r8 == 0, jnp.broadcast_to(te_row, (8, NT)),
          jnp.where(r8 == 1, jnp.broadcast_to(fr, (8, NT)),
          jnp.where(r8 == 2, jnp.broadcast_to(pr, (8, NT)),
          jnp.where(r8 == 3, jnp.broadcast_to(nxe, (8, NT)),
          jnp.where(r8 == 4, jnp.broadcast_to(hn, (8, NT)), 0.0)))))
    te_ref[...] = aux.astype(jnp.int32)


def _slots(rank, topi, cnt):
    return pl.pallas_call(
        _slots_body,
        grid=(N // NB,),
        in_specs=[
            pl.BlockSpec((NB, E), lambda i: (i, 0)),
            pl.BlockSpec((NB, E), lambda i: (i, 0)),
            pl.BlockSpec((1, E), lambda i: (0, 0)),
            pl.BlockSpec((E, 1), lambda i: (0, 0)),
        ],
        out_specs=[
            pl.BlockSpec((2, NB), lambda i: (0, i)),
            pl.BlockSpec((8, NT), lambda i: (0, 0)),
        ],
        out_shape=[
            jax.ShapeDtypeStruct((2, N), jnp.int32),
            jax.ShapeDtypeStruct((8, NT), jnp.int32),
        ],
    )(rank, topi, cnt, cnt.reshape(E, 1))


def _dispatch_scatter_sc(x1, idx_flat):
    info = plsc.get_sparse_core_info()
    nw = info.num_cores * info.num_subcores
    bpw = N // nw
    mesh = plsc.VectorSubcoreMesh(core_axis_name="c", subcore_axis_name="s")

    @functools.partial(
        pl.kernel,
        out_type=jax.ShapeDtypeStruct((S, D), jnp.float32),
        mesh=mesh,
        scratch_types=[
            pltpu.VMEM((bpw, D), jnp.float32),
            pltpu.VMEM((bpw,), jnp.int32),
            pltpu.VMEM((bpw,), jnp.int32),
            pltpu.SemaphoreType.DMA,
            pltpu.SemaphoreType.DMA,
        ],
    )
    def k(x_hbm, idx_hbm, xs_hbm, rows_v, i0_v, i1_v, sem0, sem1):
        wid = lax.axis_index("s") * info.num_cores + lax.axis_index("c")
        base = wid * bpw
        pltpu.sync_copy(x_hbm.at[pl.ds(base, bpw)], rows_v)
        pltpu.sync_copy(idx_hbm.at[pl.ds(base, bpw)], i0_v)
        pltpu.sync_copy(idx_hbm.at[pl.ds(N + base, bpw)], i1_v)
        c0 = pltpu.async_copy(rows_v, xs_hbm.at[i0_v], sem0)
        c1 = pltpu.async_copy(rows_v, xs_hbm.at[i1_v], sem1)
        c0.wait()
        c1.wait()

    return k(x1, idx_flat)


def _combine_gather_sc(xout, idx_flat):
    info = plsc.get_sparse_core_info()
    nw = info.num_cores * info.num_subcores
    bpw = N // nw
    mesh = plsc.VectorSubcoreMesh(core_axis_name="c", subcore_axis_name="s")

    @functools.partial(
        pl.kernel,
        out_type=[
            jax.ShapeDtypeStruct((N, D), jnp.float32),
            jax.ShapeDtypeStruct((N, D), jnp.float32),
        ],
        mesh=mesh,
        scratch_types=[
            pltpu.VMEM((bpw, D), jnp.float32),
            pltpu.VMEM((bpw, D), jnp.float32),
            pltpu.VMEM((bpw,), jnp.int32),
            pltpu.VMEM((bpw,), jnp.int32),
            pltpu.SemaphoreType.DMA,
            pltpu.SemaphoreType.DMA,
        ],
    )
    def k(xo_hbm, idx_hbm, g0_hbm, g1_hbm, r0_v, r1_v, i0_v, i1_v, sem0, sem1):
        wid = lax.axis_index("s") * info.num_cores + lax.axis_index("c")
        base = wid * bpw
        pltpu.sync_copy(idx_hbm.at[pl.ds(base, bpw)], i0_v)
        pltpu.sync_copy(idx_hbm.at[pl.ds(N + base, bpw)], i1_v)
        c0 = pltpu.async_copy(xo_hbm.at[i0_v], r0_v, sem0)
        c1 = pltpu.async_copy(xo_hbm.at[i1_v], r1_v, sem1)
        c0.wait()
        c1.wait()
        pltpu.sync_copy(r0_v, g0_hbm.at[pl.ds(base, bpw)])
        pltpu.sync_copy(r1_v, g1_hbm.at[pl.ds(base, bpw)])

    return k(xout, idx_flat)


def _ffn_body(te_ref, fr_ref, pr_ref, nxe_ref, hn_ref,
              xs_ref, b1_ref, b2_ref, bo_ref, w1_hbm, w2_hbm, wo_hbm,
              out_ref, w1s_ref, w2s_ref, wos_ref,
              w1b_ref, w2b_ref, wob_ref, sem):
    j = pl.program_id(0)
    i = pl.program_id(1)

    def _issue(e_, slot, jj=None):
        jb = j if jj is None else jj
        pltpu.make_async_copy(
            w1_hbm.at[e_, pl.ds(jb * FB, FB), :], w1s_ref.at[slot],
            sem.at[0, slot]).start()
        pltpu.make_async_copy(
            w2_hbm.at[e_, pl.ds(jb * FB, FB), :], w2s_ref.at[slot],
            sem.at[1, slot]).start()
        pltpu.make_async_copy(
            wo_hbm.at[e_, :, pl.ds(jb * FB, FB)], wos_ref.at[slot],
            sem.at[2, slot]).start()

    def _wait(e_, slot):
        pltpu.make_async_copy(
            w1_hbm.at[e_, pl.ds(j * FB, FB), :], w1s_ref.at[slot],
            sem.at[0, slot]).wait()
        pltpu.make_async_copy(
            w2_hbm.at[e_, pl.ds(j * FB, FB), :], w2s_ref.at[slot],
            sem.at[1, slot]).wait()
        pltpu.make_async_copy(
            wo_hbm.at[e_, :, pl.ds(j * FB, FB)], wos_ref.at[slot],
            sem.at[2, slot]).wait()

    @pl.when((i == 0) & (j == 0))
    def _():
        _issue(te_ref[0], 0)

    @pl.when(fr_ref[i] == 1)
    def _():
        slot = pr_ref[i]

        @pl.when(slot == 0)
        def _():
            _wait(te_ref[i], 0)
            w1b_ref[...] = w1s_ref[0].astype(jnp.bfloat16)
            w2b_ref[...] = w2s_ref[0].astype(jnp.bfloat16)
            wob_ref[...] = wos_ref[0].astype(jnp.bfloat16)

            @pl.when(hn_ref[i] == 1)
            def _():
                _issue(nxe_ref[i], 1)

            @pl.when((hn_ref[i] == 0) & (j < NJ - 1))
            def _():
                _issue(te_ref[0], 0, jj=j + 1)

        @pl.when(slot == 1)
        def _():
            _wait(te_ref[i], 1)
            w1b_ref[...] = w1s_ref[1].astype(jnp.bfloat16)
            w2b_ref[...] = w2s_ref[1].astype(jnp.bfloat16)
            wob_ref[...] = wos_ref[1].astype(jnp.bfloat16)

            @pl.when(hn_ref[i] == 1)
            def _():
                _issue(nxe_ref[i], 0)

            @pl.when((hn_ref[i] == 0) & (j < NJ - 1))
            def _():
                _issue(te_ref[0], 0, jj=j + 1)

    x = xs_ref[...].astype(jnp.bfloat16)
    h1 = lax.dot_general(x, w1b_ref[...], (((1,), (1,)), ((), ())),
                         preferred_element_type=jnp.float32) + b1_ref[0]
    h2 = lax.dot_general(x, w2b_ref[...], (((1,), (1,)), ((), ())),
                         preferred_element_type=jnp.float32) + b2_ref[0]
    g = (h1 / (1.0 + jnp.exp(-h1))) * h2
    g = g.astype(jnp.bfloat16)
    acc = lax.dot_general(g, wob_ref[...], (((1,), (1,)), ((), ())),
                          preferred_element_type=jnp.float32)

    @pl.when(j == 0)
    def _():
        out_ref[pl.ds(i * T, T), :] = acc + bo_ref[0]

    @pl.when(j > 0)
    def _():
        out_ref[pl.ds(i * T, T), :] += acc


def _grouped_ffn(te, fr, pr, nxe, hn, xs, W1, b1, W2, b2, Wout, bout):
    grid_spec = pltpu.PrefetchScalarGridSpec(
        num_scalar_prefetch=5,
        grid=(NJ, NT),
        in_specs=[
            pl.BlockSpec((T, D), lambda j, i, *_: (i, 0)),
            pl.BlockSpec((1, 1, FB),
                         lambda j, i, te, *_: (te[i] * NJ + j, 0, 0)),
            pl.BlockSpec((1, 1, FB),
                         lambda j, i, te, *_: (te[i] * NJ + j, 0, 0)),
            pl.BlockSpec((1, 1, D), lambda j, i, te, *_: (te[i], 0, 0)),
            pl.BlockSpec(memory_space=pltpu.MemorySpace.HBM),
            pl.BlockSpec(memory_space=pltpu.MemorySpace.HBM),
            pl.BlockSpec(memory_space=pltpu.MemorySpace.HBM),
        ],
        out_specs=pl.BlockSpec((S, D), lambda j, i, *_: (0, 0)),
        scratch_shapes=[
            pltpu.VMEM((2, FB, D), jnp.float32),
            pltpu.VMEM((2, FB, D), jnp.float32),
            pltpu.VMEM((2, D, FB), jnp.float32),
            pltpu.VMEM((FB, D), jnp.bfloat16),
            pltpu.VMEM((FB, D), jnp.bfloat16),
            pltpu.VMEM((D, FB), jnp.bfloat16),
            pltpu.SemaphoreType.DMA((3, 2)),
        ],
    )
    return pl.pallas_call(
        _ffn_body,
        grid_spec=grid_spec,
        out_shape=jax.ShapeDtypeStruct((S, D), jnp.float32),
    )(te, fr, pr, nxe, hn, xs, b1.reshape(E * NJ, 1, FB),
      b2.reshape(E * NJ, 1, FB), bout.reshape(E, 1, D), W1, W2, Wout)


def _final_body(x1_ref, g0_ref, g1_ref, tw_ref, n2w_ref, n2b_ref, o_ref):
    w0 = tw_ref[:, 0:1]
    w1 = tw_ref[:, 1:2]
    y = x1_ref[...] + w0 * g0_ref[...] + w1 * g1_ref[...]
    mu = jnp.mean(y, axis=1, keepdims=True)
    yc = y - mu
    var = jnp.mean(yc * yc, axis=1, keepdims=True)
    o_ref[...] = yc * lax.rsqrt(var + EPS) * n2w_ref[...] + n2b_ref[...]


def _final(x1, g0, g1, topw, norm2_w, norm2_b):
    return pl.pallas_call(
        _final_body,
        grid=(N // NB,),
        in_specs=[
            pl.BlockSpec((NB, D), lambda i: (i, 0)),
            pl.BlockSpec((NB, D), lambda i: (i, 0)),
            pl.BlockSpec((NB, D), lambda i: (i, 0)),
            pl.BlockSpec((NB, E), lambda i: (i, 0)),
            pl.BlockSpec((1, D), lambda i: (0, 0)),
            pl.BlockSpec((1, D), lambda i: (0, 0)),
        ],
        out_specs=pl.BlockSpec((NB, D), lambda i: (i, 0)),
        out_shape=jax.ShapeDtypeStruct((N, D), jnp.float32),
    )(x1, g0, g1, topw, norm2_w.reshape(1, D), norm2_b.reshape(1, D))


def kernel(src, in_proj_w, in_proj_b, out_proj_w, out_proj_b,
           norm1_w, norm1_b, norm2_w, norm2_b,
           gate_w, gate_b, W1, b1, W2, b2, Wout, bout):
    qkvh = _qkv(src, in_proj_w, in_proj_b)
    oh = _attention(qkvh)
    x1, topi, topw, rank, cnt = _post_attn(oh, src, out_proj_w, out_proj_b,
                                           norm1_w, norm1_b, gate_w, gate_b)
    spos, aux = _slots(rank, topi, cnt)
    idx_flat = spos.reshape(2 * N)
    xs = _dispatch_scatter_sc(x1, idx_flat)
    xout = _grouped_ffn(aux[0], aux[1], aux[2], aux[3], aux[4],
                        xs, W1, b1, W2, b2, Wout, bout)
    g0, g1 = _combine_gather_sc(xout, idx_flat)
    return _final(x1, g0, g1, topw, norm2_w, norm2_b)
